# baseline (speedup 1.0000x reference)
import functools
import math

import jax
import jax.numpy as jnp
from jax.experimental import pallas as pl
from jax.experimental.pallas import tpu as pltpu

B, L0, C_IN = 2, 2048, 64
D_MODEL, N_HEADS, E_LAYERS, D_FF = 1024, 16, 3, 4096
FACTOR, PRED_LEN = 5, 96
DH = D_MODEL // N_HEADS

_F32 = jnp.float32


def _ln(x, g, b, eps=1e-5):
    m = jnp.mean(x, axis=-1, keepdims=True)
    v = jnp.mean((x - m) ** 2, axis=-1, keepdims=True)
    return (x - m) * jax.lax.rsqrt(v + eps) * g + b


def _embed_kern(xc_ref, w_ref, pos_ref, o_ref):
    o_ref[...] = (
        jnp.dot(xc_ref[...], w_ref[...], preferred_element_type=_F32)
        + pos_ref[...]
    )


def _embed(xcat, wcat, pos, bn=256):
    n, din = xcat.shape
    d = wcat.shape[1]
    nb_l = L0 // bn
    return pl.pallas_call(
        _embed_kern,
        grid=(n // bn,),
        in_specs=[
            pl.BlockSpec((bn, din), lambda i: (i, 0)),
            pl.BlockSpec((din, d), lambda i: (0, 0)),
            pl.BlockSpec((bn, d), lambda i: (i % nb_l, 0)),
        ],
        out_specs=pl.BlockSpec((bn, d), lambda i: (i, 0)),
        out_shape=jax.ShapeDtypeStruct((n, d), _F32),
    )(xcat, wcat, pos)


_BF16 = jnp.bfloat16


def _bdot(a, b):
    return jnp.dot(a.astype(_BF16), b.astype(_BF16), preferred_element_type=_F32)


def _linear_kern(x_ref, w_ref, b_ref, o_ref):
    o_ref[...] = _bdot(x_ref[...], w_ref[...]) + b_ref[...]


def _linear(x, w, b, bn=256):
    n, din = x.shape
    d = w.shape[1]
    return pl.pallas_call(
        _linear_kern,
        grid=(n // bn,),
        in_specs=[
            pl.BlockSpec((bn, din), lambda i: (i, 0)),
            pl.BlockSpec((din, d), lambda i: (0, 0)),
            pl.BlockSpec((1, d), lambda i: (0, 0)),
        ],
        out_specs=pl.BlockSpec((bn, d), lambda i: (i, 0)),
        out_shape=jax.ShapeDtypeStruct((n, d), _F32),
    )(x, w, b.reshape(1, d))


def _measure_kern(q_ref, kt_ref, cnt_ref, m_ref, *, ns, bq):
    lq = q_ref.shape[1]
    kt = kt_ref[0]

    def mblk(i, carry):
        qb = q_ref[0, pl.ds(i * bq, bq), :]
        cb = cnt_ref[pl.ds(i * bq, bq), :]
        s = jnp.dot(qb, kt, preferred_element_type=_F32)
        mean = jnp.sum(s * cb, axis=1) / ns
        mx = jnp.max(jnp.where(cb > 0.0, s, -jnp.inf), axis=1)
        m_ref[0, 0, pl.ds(i * bq, bq)] = mx - mean
        return carry

    jax.lax.fori_loop(0, lq // bq, mblk, 0)


def _topk_kern(m_ref, t_ref, *, ns, nsp):
    bh, _, lq = m_ref.shape
    m = m_ref[:, 0, :]
    ji = jax.lax.broadcasted_iota(jnp.int32, (bh, lq), 1)
    ci = jax.lax.broadcasted_iota(jnp.int32, (bh, nsp), 1)

    def sel(t, carry):
        m, tops = carry
        mval = jnp.max(m, axis=1, keepdims=True)
        it = jnp.min(jnp.where(m >= mval, ji, lq), axis=1, keepdims=True)
        tops = jnp.where(ci == t, it, tops)
        m = jnp.where(ji == it, -jnp.inf, m)
        return m, tops

    _, tops = jax.lax.fori_loop(
        0, ns, sel, (m, jnp.full((bh, nsp), lq, jnp.int32))
    )
    t_ref[:, 0, :] = tops


def _ctx_kern(q_ref, kt_ref, v_ref, t_ref, o_ref, *, ns):
    lq = q_ref.shape[1]
    kt = kt_ref[0]
    v = v_ref[0]
    rows = [q_ref[0, pl.ds(t_ref[0, 0, n], 1), :] for n in range(ns)]
    qred = jnp.concatenate(rows, axis=0)
    scores = jnp.dot(qred, kt, preferred_element_type=_F32) * (
        1.0 / math.sqrt(DH)
    )
    smax = jnp.max(scores, axis=1, keepdims=True)
    e = jnp.exp(scores - smax)
    attn = e / jnp.sum(e, axis=1, keepdims=True)
    ctx = jnp.dot(attn, v, preferred_element_type=_F32)
    o_ref[0] = jnp.broadcast_to(jnp.mean(v, axis=0, keepdims=True), v.shape)
    for n in range(ns):
        o_ref[0, pl.ds(t_ref[0, 0, n], 1), :] = ctx[n : n + 1, :]


def _attention(q, kt, v, cnt, ns, bq=256):
    bh, lq, dh = q.shape
    nsp = 64
    m = pl.pallas_call(
        functools.partial(_measure_kern, ns=ns, bq=min(bq, lq)),
        grid=(bh,),
        in_specs=[
            pl.BlockSpec((1, lq, dh), lambda i: (i, 0, 0)),
            pl.BlockSpec((1, dh, lq), lambda i: (i, 0, 0)),
            pl.BlockSpec((lq, lq), lambda i: (0, 0)),
        ],
        out_specs=pl.BlockSpec((1, 1, lq), lambda i: (i, 0, 0)),
        out_shape=jax.ShapeDtypeStruct((bh, 1, lq), _F32),
    )(q, kt, cnt)
    tops = pl.pallas_call(
        functools.partial(_topk_kern, ns=ns, nsp=nsp),
        out_shape=jax.ShapeDtypeStruct((bh, 1, nsp), jnp.int32),
    )(m)
    return pl.pallas_call(
        functools.partial(_ctx_kern, ns=ns),
        grid=(bh,),
        in_specs=[
            pl.BlockSpec((1, lq, dh), lambda i: (i, 0, 0)),
            pl.BlockSpec((1, dh, lq), lambda i: (i, 0, 0)),
            pl.BlockSpec((1, lq, dh), lambda i: (i, 0, 0)),
            pl.BlockSpec((1, 1, nsp), lambda i: (i, 0, 0), memory_space=pltpu.SMEM),
        ],
        out_specs=pl.BlockSpec((1, lq, dh), lambda i: (i, 0, 0)),
        out_shape=jax.ShapeDtypeStruct((bh, lq, dh), _F32),
    )(q, kt, v, tops)


def _ores_kern(ctx_ref, x_ref, w_ref, b_ref, g_ref, bb_ref, xres_ref, y_ref):
    xr = x_ref[...] + _bdot(ctx_ref[...], w_ref[...]) + b_ref[...]
    xres_ref[...] = xr
    y_ref[...] = _ln(xr, g_ref[...], bb_ref[...])


def _ores(ctx, x, w, b, g, bb, bn=256):
    n, d = x.shape
    return pl.pallas_call(
        _ores_kern,
        grid=(n // bn,),
        in_specs=[
            pl.BlockSpec((bn, d), lambda i: (i, 0)),
            pl.BlockSpec((bn, d), lambda i: (i, 0)),
            pl.BlockSpec((d, d), lambda i: (0, 0)),
            pl.BlockSpec((1, d), lambda i: (0, 0)),
            pl.BlockSpec((1, d), lambda i: (0, 0)),
            pl.BlockSpec((1, d), lambda i: (0, 0)),
        ],
        out_specs=[
            pl.BlockSpec((bn, d), lambda i: (i, 0)),
            pl.BlockSpec((bn, d), lambda i: (i, 0)),
        ],
        out_shape=[
            jax.ShapeDtypeStruct((n, d), _F32),
            jax.ShapeDtypeStruct((n, d), _F32),
        ],
    )(ctx, x, w, b.reshape(1, d), g.reshape(1, d), bb.reshape(1, d))


def _ffn_kern(y_ref, xr_ref, w1_ref, b1_ref, w2_ref, b2_ref, g_ref, bb_ref, o_ref):
    h = _bdot(y_ref[...], w1_ref[...]) + b1_ref[...]
    h = 0.5 * h * (1.0 + jax.lax.erf(h * (1.0 / math.sqrt(2.0))))
    y2 = _bdot(h, w2_ref[...]) + b2_ref[...]
    o_ref[...] = _ln(xr_ref[...] + y2, g_ref[...], bb_ref[...])


def _ffn(y, xres, w1, b1, w2, b2, g, bb, bn=256):
    n, d = y.shape
    dff = w1.shape[1]
    return pl.pallas_call(
        _ffn_kern,
        grid=(n // bn,),
        in_specs=[
            pl.BlockSpec((bn, d), lambda i: (i, 0)),
            pl.BlockSpec((bn, d), lambda i: (i, 0)),
            pl.BlockSpec((d, dff), lambda i: (0, 0)),
            pl.BlockSpec((1, dff), lambda i: (0, 0)),
            pl.BlockSpec((dff, d), lambda i: (0, 0)),
            pl.BlockSpec((1, d), lambda i: (0, 0)),
            pl.BlockSpec((1, d), lambda i: (0, 0)),
            pl.BlockSpec((1, d), lambda i: (0, 0)),
        ],
        out_specs=pl.BlockSpec((bn, d), lambda i: (i, 0)),
        out_shape=jax.ShapeDtypeStruct((n, d), _F32),
    )(
        y,
        xres,
        w1,
        b1.reshape(1, dff),
        w2,
        b2.reshape(1, d),
        g.reshape(1, d),
        bb.reshape(1, d),
    )


def _distill_kern(x_ref, w_ref, b_ref, g_ref, bb_ref, o_ref, *, bl, nblk):
    j = pl.program_id(1)
    d = x_ref.shape[2]
    xb = x_ref[0, pl.ds(j * bl, bl + 4), :]
    y = (
        _bdot(xb[: bl + 2], w_ref[0])
        + _bdot(xb[1 : bl + 3], w_ref[1])
        + _bdot(xb[2 : bl + 4], w_ref[2])
        + b_ref[...]
    )
    y = y * (1.0 / math.sqrt(1.0 + 1e-5)) * g_ref[...] + bb_ref[...]
    y = jnp.where(y > 0.0, y, jnp.exp(y) - 1.0)
    ri = jax.lax.broadcasted_iota(jnp.int32, (bl + 2, 1), 0)
    edge = ((ri == 0) & (j == 0)) | ((ri == bl + 1) & (j == nblk - 1))
    y = jnp.where(edge, -jnp.inf, y)
    o_ref[0] = jnp.maximum(jnp.maximum(y[:bl], y[1 : bl + 1]), y[2 : bl + 2])


def _distill(x, w, b, g, bb, bl=256):
    bsz, l, d = x.shape
    xext = jnp.concatenate(
        [x[:, -2:, :], x, x[:, :2, :], jnp.zeros((bsz, 4, d), _F32)], axis=1
    )
    nblk = l // bl
    kern = functools.partial(_distill_kern, bl=bl, nblk=nblk)
    z = pl.pallas_call(
        kern,
        grid=(bsz, nblk),
        in_specs=[
            pl.BlockSpec((1, l + 8, d), lambda i, j: (i, 0, 0)),
            pl.BlockSpec((3, d, d), lambda i, j: (0, 0, 0)),
            pl.BlockSpec((1, d), lambda i, j: (0, 0)),
            pl.BlockSpec((1, d), lambda i, j: (0, 0)),
            pl.BlockSpec((1, d), lambda i, j: (0, 0)),
        ],
        out_specs=pl.BlockSpec((1, bl, d), lambda i, j: (i, j, 0)),
        out_shape=jax.ShapeDtypeStruct((bsz, l, d), _F32),
    )(xext, w, b.reshape(1, d), g.reshape(1, d), bb.reshape(1, d))
    return z[:, ::2, :]


def _head_kern(h_ref, x_ref, g_ref, b_ref, hw_ref, hb_ref, sw_ref, sb_ref, o_ref):
    for bi in range(B):
        h = _ln(h_ref[bi], g_ref[...], b_ref[...])
        pooled = jnp.mean(h, axis=0, keepdims=True)
        xm = jnp.mean(x_ref[bi], axis=0, keepdims=True)
        o_ref[pl.ds(bi, 1), :] = (
            jnp.dot(pooled, hw_ref[...], preferred_element_type=_F32)
            + hb_ref[...]
            + jnp.dot(xm, sw_ref[...], preferred_element_type=_F32)
            + sb_ref[...]
        )


def _head(h, x, g, b, hw, hb, sw, sb):
    _, l2, d = h.shape
    return pl.pallas_call(
        _head_kern,
        out_shape=jax.ShapeDtypeStruct((B, PRED_LEN), _F32),
    )(
        h,
        x,
        g.reshape(1, d),
        b.reshape(1, d),
        hw,
        hb.reshape(1, PRED_LEN),
        sw,
        sb.reshape(1, PRED_LEN),
    )


def _pos_emb(l, d):
    position = jnp.arange(l, dtype=_F32)[:, None]
    div = jnp.exp(
        jnp.arange(0, d, 2, dtype=_F32) * (-math.log(10000.0) / d)
    )
    pe = jnp.zeros((l, d), _F32)
    pe = pe.at[:, 0::2].set(jnp.sin(position * div))
    pe = pe.at[:, 1::2].set(jnp.cos(position * div))
    return pe


def _sample_consts(layer_i, lq):
    sample_k = min(lq, int(FACTOR * math.log(lq + 1)))
    n_top = min(lq, int(FACTOR * math.log(lq + 1)))
    key = jax.random.fold_in(jax.random.key(42), layer_i)
    idx = jax.random.randint(key, (lq, sample_k), 0, lq)
    cnt = jnp.zeros((lq, lq), _F32).at[jnp.arange(lq)[:, None], idx].add(1.0)
    return cnt, n_top


def kernel(x, params):
    p = params
    bsz, l, _ = x.shape

    xcat = jnp.concatenate(
        [jnp.roll(x, 1, axis=1), x, jnp.roll(x, -1, axis=1)], axis=-1
    ).reshape(bsz * l, 3 * C_IN)
    wcat = p["token_conv_w"].reshape(3 * C_IN, D_MODEL)
    h = _embed(xcat, wcat, _pos_emb(l, D_MODEL))

    lq = l
    for i in range(E_LAYERS):
        lp = p["layers"][i]
        cnt, n_top = _sample_consts(i, lq)
        wqkv = jnp.concatenate([lp["q_w"], lp["k_w"], lp["v_w"]], axis=1)
        bqkv = jnp.concatenate([lp["q_b"], lp["k_b"], lp["v_b"]], axis=0)
        qkv = _linear(h, wqkv, bqkv)
        qkv = qkv.reshape(bsz, lq, 3, N_HEADS, DH)
        q = qkv[:, :, 0].transpose(0, 2, 1, 3).reshape(bsz * N_HEADS, lq, DH)
        kt = qkv[:, :, 1].transpose(0, 2, 3, 1).reshape(bsz * N_HEADS, DH, lq)
        v = qkv[:, :, 2].transpose(0, 2, 1, 3).reshape(bsz * N_HEADS, lq, DH)
        ctx = _attention(q, kt, v, cnt, n_top)
        ctx = (
            ctx.reshape(bsz, N_HEADS, lq, DH)
            .transpose(0, 2, 1, 3)
            .reshape(bsz * lq, D_MODEL)
        )
        xres, y = _ores(ctx, h, lp["o_w"], lp["o_b"], lp["n1_g"], lp["n1_b"])
        h = _ffn(
            y, xres, lp["ff1_w"], lp["ff1_b"], lp["ff2_w"], lp["ff2_b"],
            lp["n2_g"], lp["n2_b"],
        )
        if i < E_LAYERS - 1:
            cp = p["convs"][i]
            h3 = h.reshape(bsz, lq, D_MODEL)
            h3 = _distill(h3, cp["conv_w"], cp["conv_b"], cp["bn_g"], cp["bn_b"])
            lq = lq // 2
            h = h3.reshape(bsz * lq, D_MODEL)

    h3 = h.reshape(bsz, lq, D_MODEL)
    return _head(
        h3, x, p["norm_g"], p["norm_b"], p["head_w"], p["head_b"],
        p["skip_w"], p["skip_b"],
    )


# speedup vs baseline: 1.1825x; 1.1825x over previous
import functools
import math

import jax
import jax.numpy as jnp
from jax.experimental import pallas as pl
from jax.experimental.pallas import tpu as pltpu

B, L0, C_IN = 2, 2048, 64
D_MODEL, N_HEADS, E_LAYERS, D_FF = 1024, 16, 3, 4096
FACTOR, PRED_LEN = 5, 96
DH = D_MODEL // N_HEADS

_F32 = jnp.float32


def _ln(x, g, b, eps=1e-5):
    m = jnp.mean(x, axis=-1, keepdims=True)
    v = jnp.mean((x - m) ** 2, axis=-1, keepdims=True)
    return (x - m) * jax.lax.rsqrt(v + eps) * g + b


def _embed_kern(xc_ref, w_ref, pos_ref, o_ref):
    o_ref[...] = (
        jnp.dot(xc_ref[...], w_ref[...], preferred_element_type=_F32)
        + pos_ref[...]
    )


def _embed(xcat, wcat, pos, bn=256):
    n, din = xcat.shape
    d = wcat.shape[1]
    nb_l = L0 // bn
    return pl.pallas_call(
        _embed_kern,
        grid=(n // bn,),
        in_specs=[
            pl.BlockSpec((bn, din), lambda i: (i, 0)),
            pl.BlockSpec((din, d), lambda i: (0, 0)),
            pl.BlockSpec((bn, d), lambda i: (i % nb_l, 0)),
        ],
        out_specs=pl.BlockSpec((bn, d), lambda i: (i, 0)),
        out_shape=jax.ShapeDtypeStruct((n, d), _F32),
    )(xcat, wcat, pos)


def _bdot(a, b):
    return jnp.dot(a, b, preferred_element_type=_F32)


def _linear_kern(x_ref, w_ref, b_ref, o_ref):
    o_ref[...] = _bdot(x_ref[...], w_ref[...]) + b_ref[...]


def _linear(x, w, b, bn=256):
    n, din = x.shape
    d = w.shape[1]
    return pl.pallas_call(
        _linear_kern,
        grid=(n // bn,),
        in_specs=[
            pl.BlockSpec((bn, din), lambda i: (i, 0)),
            pl.BlockSpec((din, d), lambda i: (0, 0)),
            pl.BlockSpec((1, d), lambda i: (0, 0)),
        ],
        out_specs=pl.BlockSpec((bn, d), lambda i: (i, 0)),
        out_shape=jax.ShapeDtypeStruct((n, d), _F32),
    )(x, w, b.reshape(1, d))


def _measure_kern(q_ref, kt_ref, cnt_ref, m_ref, *, ns, bq):
    lq = q_ref.shape[1]
    kt = kt_ref[0]

    def mblk(i, carry):
        qb = q_ref[0, pl.ds(i * bq, bq), :]
        cb = cnt_ref[pl.ds(i * bq, bq), :]
        s = jnp.dot(qb, kt, preferred_element_type=_F32)
        mean = jnp.sum(s * cb, axis=1) / ns
        mx = jnp.max(jnp.where(cb > 0.0, s, -jnp.inf), axis=1)
        m_ref[0, 0, pl.ds(i * bq, bq)] = mx - mean
        return carry

    jax.lax.fori_loop(0, lq // bq, mblk, 0)


def _topk_kern(m_ref, t_ref, *, ns, nsp):
    bh, _, lq = m_ref.shape
    m = m_ref[:, 0, :]
    ji = jax.lax.broadcasted_iota(jnp.int32, (bh, lq), 1)
    ci = jax.lax.broadcasted_iota(jnp.int32, (bh, nsp), 1)

    def sel(t, carry):
        m, tops = carry
        mval = jnp.max(m, axis=1, keepdims=True)
        it = jnp.min(jnp.where(m >= mval, ji, lq), axis=1, keepdims=True)
        tops = jnp.where(ci == t, it, tops)
        m = jnp.where(ji == it, -jnp.inf, m)
        return m, tops

    _, tops = jax.lax.fori_loop(
        0, ns, sel, (m, jnp.full((bh, nsp), lq, jnp.int32))
    )
    t_ref[:, 0, :] = tops


def _ctx_kern(q_ref, kt_ref, v_ref, t_ref, o_ref, *, ns):
    lq = q_ref.shape[1]
    kt = kt_ref[0]
    v = v_ref[0]
    rows = [q_ref[0, pl.ds(t_ref[0, 0, n], 1), :] for n in range(ns)]
    qred = jnp.concatenate(rows, axis=0)
    scores = jnp.dot(qred, kt, preferred_element_type=_F32) * (
        1.0 / math.sqrt(DH)
    )
    smax = jnp.max(scores, axis=1, keepdims=True)
    e = jnp.exp(scores - smax)
    attn = e / jnp.sum(e, axis=1, keepdims=True)
    ctx = jnp.dot(attn, v, preferred_element_type=_F32)
    o_ref[0] = jnp.broadcast_to(jnp.mean(v, axis=0, keepdims=True), v.shape)
    for n in range(ns):
        o_ref[0, pl.ds(t_ref[0, 0, n], 1), :] = ctx[n : n + 1, :]


def _attention(q, kt, v, cnt, ns, bq=256):
    bh, lq, dh = q.shape
    nsp = 64
    m = pl.pallas_call(
        functools.partial(_measure_kern, ns=ns, bq=min(bq, lq)),
        grid=(bh,),
        in_specs=[
            pl.BlockSpec((1, lq, dh), lambda i: (i, 0, 0)),
            pl.BlockSpec((1, dh, lq), lambda i: (i, 0, 0)),
            pl.BlockSpec((lq, lq), lambda i: (0, 0)),
        ],
        out_specs=pl.BlockSpec((1, 1, lq), lambda i: (i, 0, 0)),
        out_shape=jax.ShapeDtypeStruct((bh, 1, lq), _F32),
    )(q, kt, cnt)
    tops = pl.pallas_call(
        functools.partial(_topk_kern, ns=ns, nsp=nsp),
        out_shape=jax.ShapeDtypeStruct((bh, 1, nsp), jnp.int32),
    )(m)
    return pl.pallas_call(
        functools.partial(_ctx_kern, ns=ns),
        grid=(bh,),
        in_specs=[
            pl.BlockSpec((1, lq, dh), lambda i: (i, 0, 0)),
            pl.BlockSpec((1, dh, lq), lambda i: (i, 0, 0)),
            pl.BlockSpec((1, lq, dh), lambda i: (i, 0, 0)),
            pl.BlockSpec((1, 1, nsp), lambda i: (i, 0, 0), memory_space=pltpu.SMEM),
        ],
        out_specs=pl.BlockSpec((1, lq, dh), lambda i: (i, 0, 0)),
        out_shape=jax.ShapeDtypeStruct((bh, lq, dh), _F32),
    )(q, kt, v, tops)


def _ores_kern(ctx_ref, x_ref, w_ref, b_ref, g_ref, bb_ref, xres_ref, y_ref):
    xr = x_ref[...] + _bdot(ctx_ref[...], w_ref[...]) + b_ref[...]
    xres_ref[...] = xr
    y_ref[...] = _ln(xr, g_ref[...], bb_ref[...])


def _ores(ctx, x, w, b, g, bb, bn=256):
    n, d = x.shape
    return pl.pallas_call(
        _ores_kern,
        grid=(n // bn,),
        in_specs=[
            pl.BlockSpec((bn, d), lambda i: (i, 0)),
            pl.BlockSpec((bn, d), lambda i: (i, 0)),
            pl.BlockSpec((d, d), lambda i: (0, 0)),
            pl.BlockSpec((1, d), lambda i: (0, 0)),
            pl.BlockSpec((1, d), lambda i: (0, 0)),
            pl.BlockSpec((1, d), lambda i: (0, 0)),
        ],
        out_specs=[
            pl.BlockSpec((bn, d), lambda i: (i, 0)),
            pl.BlockSpec((bn, d), lambda i: (i, 0)),
        ],
        out_shape=[
            jax.ShapeDtypeStruct((n, d), _F32),
            jax.ShapeDtypeStruct((n, d), _F32),
        ],
    )(ctx, x, w, b.reshape(1, d), g.reshape(1, d), bb.reshape(1, d))


def _ffn_kern(y_ref, xr_ref, w1_ref, b1_ref, w2_ref, b2_ref, g_ref, bb_ref, o_ref):
    h = _bdot(y_ref[...], w1_ref[...]) + b1_ref[...]
    h = 0.5 * h * (1.0 + jax.lax.erf(h * (1.0 / math.sqrt(2.0))))
    y2 = _bdot(h, w2_ref[...]) + b2_ref[...]
    o_ref[...] = _ln(xr_ref[...] + y2, g_ref[...], bb_ref[...])


def _ffn(y, xres, w1, b1, w2, b2, g, bb, bn=256):
    n, d = y.shape
    dff = w1.shape[1]
    return pl.pallas_call(
        _ffn_kern,
        grid=(n // bn,),
        in_specs=[
            pl.BlockSpec((bn, d), lambda i: (i, 0)),
            pl.BlockSpec((bn, d), lambda i: (i, 0)),
            pl.BlockSpec((d, dff), lambda i: (0, 0)),
            pl.BlockSpec((1, dff), lambda i: (0, 0)),
            pl.BlockSpec((dff, d), lambda i: (0, 0)),
            pl.BlockSpec((1, d), lambda i: (0, 0)),
            pl.BlockSpec((1, d), lambda i: (0, 0)),
            pl.BlockSpec((1, d), lambda i: (0, 0)),
        ],
        out_specs=pl.BlockSpec((bn, d), lambda i: (i, 0)),
        out_shape=jax.ShapeDtypeStruct((n, d), _F32),
    )(
        y,
        xres,
        w1,
        b1.reshape(1, dff),
        w2,
        b2.reshape(1, d),
        g.reshape(1, d),
        bb.reshape(1, d),
    )


def _distill_kern(x_ref, w_ref, b_ref, g_ref, bb_ref, o_ref, *, bl, nblk):
    j = pl.program_id(1)
    d = x_ref.shape[2]
    xb = x_ref[0, pl.ds(j * bl, bl + 4), :]
    y = (
        _bdot(xb[: bl + 2], w_ref[0])
        + _bdot(xb[1 : bl + 3], w_ref[1])
        + _bdot(xb[2 : bl + 4], w_ref[2])
        + b_ref[...]
    )
    y = y * (1.0 / math.sqrt(1.0 + 1e-5)) * g_ref[...] + bb_ref[...]
    y = jnp.where(y > 0.0, y, jnp.exp(y) - 1.0)
    ri = jax.lax.broadcasted_iota(jnp.int32, (bl + 2, 1), 0)
    edge = ((ri == 0) & (j == 0)) | ((ri == bl + 1) & (j == nblk - 1))
    y = jnp.where(edge, -jnp.inf, y)
    o_ref[0] = jnp.maximum(jnp.maximum(y[:bl], y[1 : bl + 1]), y[2 : bl + 2])


def _distill(x, w, b, g, bb, bl=256):
    bsz, l, d = x.shape
    xext = jnp.concatenate(
        [x[:, -2:, :], x, x[:, :2, :], jnp.zeros((bsz, 4, d), _F32)], axis=1
    )
    nblk = l // bl
    kern = functools.partial(_distill_kern, bl=bl, nblk=nblk)
    z = pl.pallas_call(
        kern,
        grid=(bsz, nblk),
        in_specs=[
            pl.BlockSpec((1, l + 8, d), lambda i, j: (i, 0, 0)),
            pl.BlockSpec((3, d, d), lambda i, j: (0, 0, 0)),
            pl.BlockSpec((1, d), lambda i, j: (0, 0)),
            pl.BlockSpec((1, d), lambda i, j: (0, 0)),
            pl.BlockSpec((1, d), lambda i, j: (0, 0)),
        ],
        out_specs=pl.BlockSpec((1, bl, d), lambda i, j: (i, j, 0)),
        out_shape=jax.ShapeDtypeStruct((bsz, l, d), _F32),
    )(xext, w, b.reshape(1, d), g.reshape(1, d), bb.reshape(1, d))
    return z[:, ::2, :]


def _head_kern(h_ref, x_ref, g_ref, b_ref, hw_ref, hb_ref, sw_ref, sb_ref, o_ref):
    for bi in range(B):
        h = _ln(h_ref[bi], g_ref[...], b_ref[...])
        pooled = jnp.mean(h, axis=0, keepdims=True)
        xm = jnp.mean(x_ref[bi], axis=0, keepdims=True)
        o_ref[pl.ds(bi, 1), :] = (
            jnp.dot(pooled, hw_ref[...], preferred_element_type=_F32)
            + hb_ref[...]
            + jnp.dot(xm, sw_ref[...], preferred_element_type=_F32)
            + sb_ref[...]
        )


def _head(h, x, g, b, hw, hb, sw, sb):
    _, l2, d = h.shape
    return pl.pallas_call(
        _head_kern,
        out_shape=jax.ShapeDtypeStruct((B, PRED_LEN), _F32),
    )(
        h,
        x,
        g.reshape(1, d),
        b.reshape(1, d),
        hw,
        hb.reshape(1, PRED_LEN),
        sw,
        sb.reshape(1, PRED_LEN),
    )


def _cnt_kern(idx_ref, c_ref, *, ns):
    bq, lq = c_ref.shape
    ji = jax.lax.broadcasted_iota(jnp.int32, (bq, lq), 1)
    idxb = idx_ref[...]
    c = jnp.zeros((bq, lq), _F32)
    for s in range(ns):
        c += (idxb[:, s : s + 1] == ji).astype(_F32)
    c_ref[...] = c


def _cnt_build(idx, lq, bq=256):
    ns = idx.shape[1]
    return pl.pallas_call(
        functools.partial(_cnt_kern, ns=ns),
        grid=(lq // bq,),
        in_specs=[pl.BlockSpec((bq, ns), lambda i: (i, 0))],
        out_specs=pl.BlockSpec((bq, lq), lambda i: (i, 0)),
        out_shape=jax.ShapeDtypeStruct((lq, lq), _F32),
    )(idx)


def _pos_emb(l, d):
    position = jnp.arange(l, dtype=_F32)[:, None]
    div = jnp.exp(
        jnp.arange(0, d, 2, dtype=_F32) * (-math.log(10000.0) / d)
    )
    pe = jnp.zeros((l, d), _F32)
    pe = pe.at[:, 0::2].set(jnp.sin(position * div))
    pe = pe.at[:, 1::2].set(jnp.cos(position * div))
    return pe


def _sample_consts(layer_i, lq):
    sample_k = min(lq, int(FACTOR * math.log(lq + 1)))
    n_top = min(lq, int(FACTOR * math.log(lq + 1)))
    key = jax.random.fold_in(jax.random.key(42), layer_i)
    idx = jax.random.randint(key, (lq, sample_k), 0, lq)
    cnt = _cnt_build(idx, lq)
    return cnt, n_top


def kernel(x, params):
    p = params
    bsz, l, _ = x.shape

    xcat = jnp.concatenate(
        [jnp.roll(x, 1, axis=1), x, jnp.roll(x, -1, axis=1)], axis=-1
    ).reshape(bsz * l, 3 * C_IN)
    wcat = p["token_conv_w"].reshape(3 * C_IN, D_MODEL)
    h = _embed(xcat, wcat, _pos_emb(l, D_MODEL))

    lq = l
    for i in range(E_LAYERS):
        lp = p["layers"][i]
        cnt, n_top = _sample_consts(i, lq)
        wqkv = jnp.concatenate([lp["q_w"], lp["k_w"], lp["v_w"]], axis=1)
        bqkv = jnp.concatenate([lp["q_b"], lp["k_b"], lp["v_b"]], axis=0)
        qkv = _linear(h, wqkv, bqkv)
        qkv = qkv.reshape(bsz, lq, 3, N_HEADS, DH)
        q = qkv[:, :, 0].transpose(0, 2, 1, 3).reshape(bsz * N_HEADS, lq, DH)
        kt = qkv[:, :, 1].transpose(0, 2, 3, 1).reshape(bsz * N_HEADS, DH, lq)
        v = qkv[:, :, 2].transpose(0, 2, 1, 3).reshape(bsz * N_HEADS, lq, DH)
        ctx = _attention(q, kt, v, cnt, n_top)
        ctx = (
            ctx.reshape(bsz, N_HEADS, lq, DH)
            .transpose(0, 2, 1, 3)
            .reshape(bsz * lq, D_MODEL)
        )
        xres, y = _ores(ctx, h, lp["o_w"], lp["o_b"], lp["n1_g"], lp["n1_b"])
        h = _ffn(
            y, xres, lp["ff1_w"], lp["ff1_b"], lp["ff2_w"], lp["ff2_b"],
            lp["n2_g"], lp["n2_b"],
        )
        if i < E_LAYERS - 1:
            cp = p["convs"][i]
            h3 = h.reshape(bsz, lq, D_MODEL)
            h3 = _distill(h3, cp["conv_w"], cp["conv_b"], cp["bn_g"], cp["bn_b"])
            lq = lq // 2
            h = h3.reshape(bsz * lq, D_MODEL)

    h3 = h.reshape(bsz, lq, D_MODEL)
    return _head(
        h3, x, p["norm_g"], p["norm_b"], p["head_w"], p["head_b"],
        p["skip_w"], p["skip_b"],
    )


# speedup vs baseline: 1.1930x; 1.0089x over previous
import functools
import math

import jax
import jax.numpy as jnp
from jax.experimental import pallas as pl
from jax.experimental.pallas import tpu as pltpu

B, L0, C_IN = 2, 2048, 64
D_MODEL, N_HEADS, E_LAYERS, D_FF = 1024, 16, 3, 4096
FACTOR, PRED_LEN = 5, 96
DH = D_MODEL // N_HEADS

_F32 = jnp.float32


def _ln(x, g, b, eps=1e-5):
    m = jnp.mean(x, axis=-1, keepdims=True)
    v = jnp.mean((x - m) ** 2, axis=-1, keepdims=True)
    return (x - m) * jax.lax.rsqrt(v + eps) * g + b


def _embed_kern(xc_ref, w_ref, pos_ref, o_ref):
    o_ref[...] = (
        jnp.dot(xc_ref[...], w_ref[...], preferred_element_type=_F32)
        + pos_ref[...]
    )


def _embed(xcat, wcat, pos, bn=256):
    n, din = xcat.shape
    d = wcat.shape[1]
    nb_l = L0 // bn
    return pl.pallas_call(
        _embed_kern,
        grid=(n // bn,),
        in_specs=[
            pl.BlockSpec((bn, din), lambda i: (i, 0)),
            pl.BlockSpec((din, d), lambda i: (0, 0)),
            pl.BlockSpec((bn, d), lambda i: (i % nb_l, 0)),
        ],
        out_specs=pl.BlockSpec((bn, d), lambda i: (i, 0)),
        out_shape=jax.ShapeDtypeStruct((n, d), _F32),
    )(xcat, wcat, pos)


def _bdot(a, b):
    return jnp.dot(a, b, preferred_element_type=_F32)


def _linear_kern(x_ref, w_ref, b_ref, o_ref):
    o_ref[...] = _bdot(x_ref[...], w_ref[...]) + b_ref[...]


def _linear(x, w, b, bn=256):
    n, din = x.shape
    d = w.shape[1]
    return pl.pallas_call(
        _linear_kern,
        grid=(n // bn,),
        in_specs=[
            pl.BlockSpec((bn, din), lambda i: (i, 0)),
            pl.BlockSpec((din, d), lambda i: (0, 0)),
            pl.BlockSpec((1, d), lambda i: (0, 0)),
        ],
        out_specs=pl.BlockSpec((bn, d), lambda i: (i, 0)),
        out_shape=jax.ShapeDtypeStruct((n, d), _F32),
    )(x, w, b.reshape(1, d))


def _measure_kern(q_ref, kt_ref, cnt_ref, m_ref, *, ns, bq):
    lq = q_ref.shape[1]
    kt = kt_ref[0]

    def mblk(i, carry):
        qb = q_ref[0, pl.ds(i * bq, bq), :]
        cb = cnt_ref[pl.ds(i * bq, bq), :]
        s = jnp.dot(qb, kt, preferred_element_type=_F32)
        mean = jnp.sum(s * cb, axis=1) / ns
        mx = jnp.max(jnp.where(cb > 0.0, s, -jnp.inf), axis=1)
        m_ref[0, 0, pl.ds(i * bq, bq)] = mx - mean
        return carry

    jax.lax.fori_loop(0, lq // bq, mblk, 0)


def _topk_kern(m_ref, t_ref, *, ns, nsp):
    bh, _, lq = m_ref.shape
    m = m_ref[:, 0, :]
    ji = jax.lax.broadcasted_iota(jnp.int32, (bh, lq), 1)
    ci = jax.lax.broadcasted_iota(jnp.int32, (bh, nsp), 1)

    def sel(t, carry):
        m, tops = carry
        mval = jnp.max(m, axis=1, keepdims=True)
        it = jnp.min(jnp.where(m >= mval, ji, lq), axis=1, keepdims=True)
        tops = jnp.where(ci == t, it, tops)
        m = jnp.where(ji == it, -jnp.inf, m)
        return m, tops

    _, tops = jax.lax.fori_loop(
        0, ns, sel, (m, jnp.full((bh, nsp), lq, jnp.int32))
    )
    t_ref[:, 0, :] = tops


def _ctx_kern(q_ref, kt_ref, v_ref, t_ref, o_ref, *, ns):
    lq = q_ref.shape[1]
    kt = kt_ref[0]
    v = v_ref[0]
    rows = [q_ref[0, pl.ds(t_ref[0, 0, n], 1), :] for n in range(ns)]
    qred = jnp.concatenate(rows, axis=0)
    scores = jnp.dot(qred, kt, preferred_element_type=_F32) * (
        1.0 / math.sqrt(DH)
    )
    smax = jnp.max(scores, axis=1, keepdims=True)
    e = jnp.exp(scores - smax)
    attn = e / jnp.sum(e, axis=1, keepdims=True)
    ctx = jnp.dot(attn, v, preferred_element_type=_F32)
    o_ref[0] = jnp.broadcast_to(jnp.mean(v, axis=0, keepdims=True), v.shape)
    for n in range(ns):
        o_ref[0, pl.ds(t_ref[0, 0, n], 1), :] = ctx[n : n + 1, :]


def _attention(q, kt, v, cnt, ns, bq=256):
    bh, lq, dh = q.shape
    nsp = 64
    m = pl.pallas_call(
        functools.partial(_measure_kern, ns=ns, bq=min(bq, lq)),
        grid=(bh,),
        in_specs=[
            pl.BlockSpec((1, lq, dh), lambda i: (i, 0, 0)),
            pl.BlockSpec((1, dh, lq), lambda i: (i, 0, 0)),
            pl.BlockSpec((lq, lq), lambda i: (0, 0)),
        ],
        out_specs=pl.BlockSpec((1, 1, lq), lambda i: (i, 0, 0)),
        out_shape=jax.ShapeDtypeStruct((bh, 1, lq), _F32),
    )(q, kt, cnt)
    tops = pl.pallas_call(
        functools.partial(_topk_kern, ns=ns, nsp=nsp),
        out_shape=jax.ShapeDtypeStruct((bh, 1, nsp), jnp.int32),
    )(m)
    return pl.pallas_call(
        functools.partial(_ctx_kern, ns=ns),
        grid=(bh,),
        in_specs=[
            pl.BlockSpec((1, lq, dh), lambda i: (i, 0, 0)),
            pl.BlockSpec((1, dh, lq), lambda i: (i, 0, 0)),
            pl.BlockSpec((1, lq, dh), lambda i: (i, 0, 0)),
            pl.BlockSpec((1, 1, nsp), lambda i: (i, 0, 0), memory_space=pltpu.SMEM),
        ],
        out_specs=pl.BlockSpec((1, lq, dh), lambda i: (i, 0, 0)),
        out_shape=jax.ShapeDtypeStruct((bh, lq, dh), _F32),
    )(q, kt, v, tops)


def _ores_kern(ctx_ref, x_ref, w_ref, b_ref, g_ref, bb_ref, xres_ref, y_ref):
    xr = x_ref[...] + _bdot(ctx_ref[...], w_ref[...]) + b_ref[...]
    xres_ref[...] = xr
    y_ref[...] = _ln(xr, g_ref[...], bb_ref[...])


def _ores(ctx, x, w, b, g, bb, bn=256):
    n, d = x.shape
    return pl.pallas_call(
        _ores_kern,
        grid=(n // bn,),
        in_specs=[
            pl.BlockSpec((bn, d), lambda i: (i, 0)),
            pl.BlockSpec((bn, d), lambda i: (i, 0)),
            pl.BlockSpec((d, d), lambda i: (0, 0)),
            pl.BlockSpec((1, d), lambda i: (0, 0)),
            pl.BlockSpec((1, d), lambda i: (0, 0)),
            pl.BlockSpec((1, d), lambda i: (0, 0)),
        ],
        out_specs=[
            pl.BlockSpec((bn, d), lambda i: (i, 0)),
            pl.BlockSpec((bn, d), lambda i: (i, 0)),
        ],
        out_shape=[
            jax.ShapeDtypeStruct((n, d), _F32),
            jax.ShapeDtypeStruct((n, d), _F32),
        ],
    )(ctx, x, w, b.reshape(1, d), g.reshape(1, d), bb.reshape(1, d))


def _ffn_kern(y_ref, xr_ref, w1_ref, b1_ref, w2_ref, b2_ref, g_ref, bb_ref, o_ref):
    h = _bdot(y_ref[...], w1_ref[...]) + b1_ref[...]
    h = 0.5 * h * (1.0 + jax.lax.erf(h * (1.0 / math.sqrt(2.0))))
    y2 = _bdot(h, w2_ref[...]) + b2_ref[...]
    o_ref[...] = _ln(xr_ref[...] + y2, g_ref[...], bb_ref[...])


def _ffn(y, xres, w1, b1, w2, b2, g, bb, bn=256):
    n, d = y.shape
    dff = w1.shape[1]
    return pl.pallas_call(
        _ffn_kern,
        grid=(n // bn,),
        in_specs=[
            pl.BlockSpec((bn, d), lambda i: (i, 0)),
            pl.BlockSpec((bn, d), lambda i: (i, 0)),
            pl.BlockSpec((d, dff), lambda i: (0, 0)),
            pl.BlockSpec((1, dff), lambda i: (0, 0)),
            pl.BlockSpec((dff, d), lambda i: (0, 0)),
            pl.BlockSpec((1, d), lambda i: (0, 0)),
            pl.BlockSpec((1, d), lambda i: (0, 0)),
            pl.BlockSpec((1, d), lambda i: (0, 0)),
        ],
        out_specs=pl.BlockSpec((bn, d), lambda i: (i, 0)),
        out_shape=jax.ShapeDtypeStruct((n, d), _F32),
    )(
        y,
        xres,
        w1,
        b1.reshape(1, dff),
        w2,
        b2.reshape(1, d),
        g.reshape(1, d),
        bb.reshape(1, d),
    )


def _distill_kern(x_ref, w_ref, b_ref, g_ref, bb_ref, o_ref, *, bl, nblk):
    j = pl.program_id(1)
    d = x_ref.shape[2]
    xb = x_ref[0, pl.ds(j * bl, bl + 4), :]
    y = (
        _bdot(xb[: bl + 2], w_ref[0])
        + _bdot(xb[1 : bl + 3], w_ref[1])
        + _bdot(xb[2 : bl + 4], w_ref[2])
        + b_ref[...]
    )
    y = y * (1.0 / math.sqrt(1.0 + 1e-5)) * g_ref[...] + bb_ref[...]
    y = jnp.where(y > 0.0, y, jnp.exp(y) - 1.0)
    ri = jax.lax.broadcasted_iota(jnp.int32, (bl + 2, 1), 0)
    edge = ((ri == 0) & (j == 0)) | ((ri == bl + 1) & (j == nblk - 1))
    y = jnp.where(edge, -jnp.inf, y)
    o_ref[0] = jnp.maximum(jnp.maximum(y[:bl], y[1 : bl + 1]), y[2 : bl + 2])


def _distill(x, w, b, g, bb, bl=256):
    bsz, l, d = x.shape
    xext = jnp.concatenate(
        [x[:, -2:, :], x, x[:, :2, :], jnp.zeros((bsz, 4, d), _F32)], axis=1
    )
    nblk = l // bl
    kern = functools.partial(_distill_kern, bl=bl, nblk=nblk)
    z = pl.pallas_call(
        kern,
        grid=(bsz, nblk),
        in_specs=[
            pl.BlockSpec((1, l + 8, d), lambda i, j: (i, 0, 0)),
            pl.BlockSpec((3, d, d), lambda i, j: (0, 0, 0)),
            pl.BlockSpec((1, d), lambda i, j: (0, 0)),
            pl.BlockSpec((1, d), lambda i, j: (0, 0)),
            pl.BlockSpec((1, d), lambda i, j: (0, 0)),
        ],
        out_specs=pl.BlockSpec((1, bl, d), lambda i, j: (i, j, 0)),
        out_shape=jax.ShapeDtypeStruct((bsz, l, d), _F32),
    )(xext, w, b.reshape(1, d), g.reshape(1, d), bb.reshape(1, d))
    return z[:, ::2, :]


def _head_kern(h_ref, x_ref, g_ref, b_ref, hw_ref, hb_ref, sw_ref, sb_ref, o_ref):
    for bi in range(B):
        h = _ln(h_ref[bi], g_ref[...], b_ref[...])
        pooled = jnp.mean(h, axis=0, keepdims=True)
        xm = jnp.mean(x_ref[bi], axis=0, keepdims=True)
        o_ref[pl.ds(bi, 1), :] = (
            jnp.dot(pooled, hw_ref[...], preferred_element_type=_F32)
            + hb_ref[...]
            + jnp.dot(xm, sw_ref[...], preferred_element_type=_F32)
            + sb_ref[...]
        )


def _head(h, x, g, b, hw, hb, sw, sb):
    _, l2, d = h.shape
    return pl.pallas_call(
        _head_kern,
        out_shape=jax.ShapeDtypeStruct((B, PRED_LEN), _F32),
    )(
        h,
        x,
        g.reshape(1, d),
        b.reshape(1, d),
        hw,
        hb.reshape(1, PRED_LEN),
        sw,
        sb.reshape(1, PRED_LEN),
    )


def _cnt_kern(idx_ref, c_ref, *, ns):
    bq, lq = c_ref.shape
    ji = jax.lax.broadcasted_iota(jnp.int32, (bq, lq), 1)
    idxb = idx_ref[...]
    c = jnp.zeros((bq, lq), _F32)
    for s in range(ns):
        c += (idxb[:, s : s + 1] == ji).astype(_F32)
    c_ref[...] = c


def _cnt_build(idx, lq, bq=256):
    ns = idx.shape[1]
    return pl.pallas_call(
        functools.partial(_cnt_kern, ns=ns),
        grid=(lq // bq,),
        in_specs=[pl.BlockSpec((bq, ns), lambda i: (i, 0))],
        out_specs=pl.BlockSpec((bq, lq), lambda i: (i, 0)),
        out_shape=jax.ShapeDtypeStruct((lq, lq), _F32),
    )(idx)


def _pos_emb(l, d):
    position = jnp.arange(l, dtype=_F32)[:, None]
    div = jnp.exp(
        jnp.arange(0, d, 2, dtype=_F32) * (-math.log(10000.0) / d)
    )
    ang = position * div
    return jnp.stack([jnp.sin(ang), jnp.cos(ang)], axis=-1).reshape(l, d)


def _sample_consts(layer_i, lq):
    sample_k = min(lq, int(FACTOR * math.log(lq + 1)))
    n_top = min(lq, int(FACTOR * math.log(lq + 1)))
    key = jax.random.fold_in(jax.random.key(42), layer_i)
    idx = jax.random.randint(key, (lq, sample_k), 0, lq)
    cnt = _cnt_build(idx, lq)
    return cnt, n_top


def kernel(x, params):
    p = params
    bsz, l, _ = x.shape

    xcat = jnp.concatenate(
        [jnp.roll(x, 1, axis=1), x, jnp.roll(x, -1, axis=1)], axis=-1
    ).reshape(bsz * l, 3 * C_IN)
    wcat = p["token_conv_w"].reshape(3 * C_IN, D_MODEL)
    h = _embed(xcat, wcat, _pos_emb(l, D_MODEL))

    lq = l
    for i in range(E_LAYERS):
        lp = p["layers"][i]
        cnt, n_top = _sample_consts(i, lq)
        wqkv = jnp.concatenate([lp["q_w"], lp["k_w"], lp["v_w"]], axis=1)
        bqkv = jnp.concatenate([lp["q_b"], lp["k_b"], lp["v_b"]], axis=0)
        qkv = _linear(h, wqkv, bqkv)
        qkv = qkv.reshape(bsz, lq, 3, N_HEADS, DH)
        q = qkv[:, :, 0].transpose(0, 2, 1, 3).reshape(bsz * N_HEADS, lq, DH)
        kt = qkv[:, :, 1].transpose(0, 2, 3, 1).reshape(bsz * N_HEADS, DH, lq)
        v = qkv[:, :, 2].transpose(0, 2, 1, 3).reshape(bsz * N_HEADS, lq, DH)
        ctx = _attention(q, kt, v, cnt, n_top)
        ctx = (
            ctx.reshape(bsz, N_HEADS, lq, DH)
            .transpose(0, 2, 1, 3)
            .reshape(bsz * lq, D_MODEL)
        )
        xres, y = _ores(ctx, h, lp["o_w"], lp["o_b"], lp["n1_g"], lp["n1_b"])
        h = _ffn(
            y, xres, lp["ff1_w"], lp["ff1_b"], lp["ff2_w"], lp["ff2_b"],
            lp["n2_g"], lp["n2_b"],
        )
        if i < E_LAYERS - 1:
            cp = p["convs"][i]
            h3 = h.reshape(bsz, lq, D_MODEL)
            h3 = _distill(h3, cp["conv_w"], cp["conv_b"], cp["bn_g"], cp["bn_b"])
            lq = lq // 2
            h = h3.reshape(bsz * lq, D_MODEL)

    h3 = h.reshape(bsz, lq, D_MODEL)
    return _head(
        h3, x, p["norm_g"], p["norm_b"], p["head_w"], p["head_b"],
        p["skip_w"], p["skip_b"],
    )


# speedup vs baseline: 1.6969x; 1.4224x over previous
import functools
import math

import jax
import jax.numpy as jnp
from jax.experimental import pallas as pl
from jax.experimental.pallas import tpu as pltpu

B, L0, C_IN = 2, 2048, 64
D_MODEL, N_HEADS, E_LAYERS, D_FF = 1024, 16, 3, 4096
FACTOR, PRED_LEN = 5, 96
DH = D_MODEL // N_HEADS

_F32 = jnp.float32


def _ln(x, g, b, eps=1e-5):
    m = jnp.mean(x, axis=-1, keepdims=True)
    v = jnp.mean((x - m) ** 2, axis=-1, keepdims=True)
    return (x - m) * jax.lax.rsqrt(v + eps) * g + b


def _embed_kern(xc_ref, w_ref, pos_ref, o_ref):
    o_ref[...] = (
        jnp.dot(xc_ref[...], w_ref[...], preferred_element_type=_F32)
        + pos_ref[...]
    )


def _embed(xcat, wcat, pos, bn=256):
    n, din = xcat.shape
    d = wcat.shape[1]
    nb_l = L0 // bn
    return pl.pallas_call(
        _embed_kern,
        grid=(n // bn,),
        in_specs=[
            pl.BlockSpec((bn, din), lambda i: (i, 0)),
            pl.BlockSpec((din, d), lambda i: (0, 0)),
            pl.BlockSpec((bn, d), lambda i: (i % nb_l, 0)),
        ],
        out_specs=pl.BlockSpec((bn, d), lambda i: (i, 0)),
        out_shape=jax.ShapeDtypeStruct((n, d), _F32),
    )(xcat, wcat, pos)


def _bdot(a, b):
    return jnp.dot(a, b, preferred_element_type=_F32)


def _linear_kern(x_ref, w_ref, b_ref, o_ref):
    o_ref[...] = _bdot(x_ref[...], w_ref[...]) + b_ref[...]


def _ntdot(a, b):
    return jax.lax.dot_general(
        a, b, (((1,), (1,)), ((), ())), preferred_element_type=_F32
    )


def _qkv_kern(x_ref, wq_ref, wk_ref, wv_ref, b_ref, q_ref, k_ref, v_ref):
    x = x_ref[...]
    d = wq_ref.shape[1]
    q_ref[...] = _bdot(x, wq_ref[...]) + b_ref[0:1, :]
    k_ref[...] = _bdot(x, wk_ref[...]) + b_ref[1:2, :]
    v_ref[...] = _bdot(x, wv_ref[...]) + b_ref[2:3, :]


def _qkv(x, wq, wk, wv, b3, bn=256):
    n, d = x.shape
    out = jax.ShapeDtypeStruct((n, d), _F32)
    return pl.pallas_call(
        _qkv_kern,
        grid=(n // bn,),
        in_specs=[
            pl.BlockSpec((bn, d), lambda i: (i, 0)),
            pl.BlockSpec((d, d), lambda i: (0, 0)),
            pl.BlockSpec((d, d), lambda i: (0, 0)),
            pl.BlockSpec((d, d), lambda i: (0, 0)),
            pl.BlockSpec((3, d), lambda i: (0, 0)),
        ],
        out_specs=[
            pl.BlockSpec((bn, d), lambda i: (i, 0)),
            pl.BlockSpec((bn, d), lambda i: (i, 0)),
            pl.BlockSpec((bn, d), lambda i: (i, 0)),
        ],
        out_shape=[out, out, out],
    )(x, wq, wk, wv, b3)


def _linear(x, w, b, bn=256):
    n, din = x.shape
    d = w.shape[1]
    return pl.pallas_call(
        _linear_kern,
        grid=(n // bn,),
        in_specs=[
            pl.BlockSpec((bn, din), lambda i: (i, 0)),
            pl.BlockSpec((din, d), lambda i: (0, 0)),
            pl.BlockSpec((1, d), lambda i: (0, 0)),
        ],
        out_specs=pl.BlockSpec((bn, d), lambda i: (i, 0)),
        out_shape=jax.ShapeDtypeStruct((n, d), _F32),
    )(x, w, b.reshape(1, d))


def _measure_kern(q_ref, k_ref, cnt_ref, m_ref, *, ns, bq):
    lq = q_ref.shape[1]
    for hh in range(2):
        c0, c1 = hh * DH, (hh + 1) * DH
        kh = k_ref[0, :, c0:c1]

        def mblk(i, carry):
            qb = q_ref[0, pl.ds(i * bq, bq), c0:c1]
            cb = cnt_ref[pl.ds(i * bq, bq), :]
            s = _ntdot(qb, kh)
            mean = jnp.sum(s * cb, axis=1) / ns
            mx = jnp.max(jnp.where(cb > 0.0, s, -jnp.inf), axis=1)
            m_ref[hh, 0, pl.ds(i * bq, bq)] = mx - mean
            return carry

        jax.lax.fori_loop(0, lq // bq, mblk, 0)


def _topk_kern(m_ref, t_ref, *, ns, nsp):
    bh, _, lq = m_ref.shape
    m = m_ref[:, 0, :]
    ji = jax.lax.broadcasted_iota(jnp.int32, (bh, lq), 1)
    ci = jax.lax.broadcasted_iota(jnp.int32, (bh, nsp), 1)

    def sel(t, carry):
        m, tops = carry
        mval = jnp.max(m, axis=1, keepdims=True)
        it = jnp.min(jnp.where(m >= mval, ji, lq), axis=1, keepdims=True)
        tops = jnp.where(ci == t, it, tops)
        m = jnp.where(ji == it, -jnp.inf, m)
        return m, tops

    _, tops = jax.lax.fori_loop(
        0, ns, sel, (m, jnp.full((bh, nsp), lq, jnp.int32))
    )
    t_ref[:, 0, :] = tops


def _ctx_kern(q_ref, k_ref, v_ref, t_ref, o_ref, *, ns):
    lq = q_ref.shape[1]
    for hh in range(2):
        c0, c1 = hh * DH, (hh + 1) * DH
        kh = k_ref[0, :, c0:c1]
        vh = v_ref[0, :, c0:c1]
        rows = [q_ref[0, pl.ds(t_ref[hh, 0, n], 1), c0:c1] for n in range(ns)]
        qred = jnp.concatenate(rows, axis=0)
        scores = _ntdot(qred, kh) * (1.0 / math.sqrt(DH))
        smax = jnp.max(scores, axis=1, keepdims=True)
        e = jnp.exp(scores - smax)
        attn = e / jnp.sum(e, axis=1, keepdims=True)
        ctx = jnp.dot(attn, vh, preferred_element_type=_F32)
        o_ref[0, :, c0:c1] = jnp.broadcast_to(
            jnp.mean(vh, axis=0, keepdims=True), vh.shape
        )
        for n in range(ns):
            o_ref[0, pl.ds(t_ref[hh, 0, n], 1), c0:c1] = ctx[n : n + 1, :]


def _attention(q, k, v, cnt, ns, bq=256):
    bsz, lq, d = q.shape
    ng = N_HEADS // 2
    bh = bsz * N_HEADS
    nsp = 64
    m = pl.pallas_call(
        functools.partial(_measure_kern, ns=ns, bq=min(bq, lq)),
        grid=(bsz, ng),
        in_specs=[
            pl.BlockSpec((1, lq, 2 * DH), lambda b, g: (b, 0, g)),
            pl.BlockSpec((1, lq, 2 * DH), lambda b, g: (b, 0, g)),
            pl.BlockSpec((lq, lq), lambda b, g: (0, 0)),
        ],
        out_specs=pl.BlockSpec((2, 1, lq), lambda b, g: (b * ng + g, 0, 0)),
        out_shape=jax.ShapeDtypeStruct((bh, 1, lq), _F32),
    )(q, k, cnt)
    tops = pl.pallas_call(
        functools.partial(_topk_kern, ns=ns, nsp=nsp),
        out_shape=jax.ShapeDtypeStruct((bh, 1, nsp), jnp.int32),
    )(m)
    return pl.pallas_call(
        functools.partial(_ctx_kern, ns=ns),
        grid=(bsz, ng),
        in_specs=[
            pl.BlockSpec((1, lq, 2 * DH), lambda b, g: (b, 0, g)),
            pl.BlockSpec((1, lq, 2 * DH), lambda b, g: (b, 0, g)),
            pl.BlockSpec((1, lq, 2 * DH), lambda b, g: (b, 0, g)),
            pl.BlockSpec(
                (2, 1, nsp), lambda b, g: (b * ng + g, 0, 0), memory_space=pltpu.SMEM
            ),
        ],
        out_specs=pl.BlockSpec((1, lq, 2 * DH), lambda b, g: (b, 0, g)),
        out_shape=jax.ShapeDtypeStruct((bsz, lq, d), _F32),
    )(q, k, v, tops)


def _ores_kern(ctx_ref, x_ref, w_ref, b_ref, g_ref, bb_ref, xres_ref, y_ref):
    xr = x_ref[...] + _bdot(ctx_ref[...], w_ref[...]) + b_ref[...]
    xres_ref[...] = xr
    y_ref[...] = _ln(xr, g_ref[...], bb_ref[...])


def _ores(ctx, x, w, b, g, bb, bn=256):
    n, d = x.shape
    return pl.pallas_call(
        _ores_kern,
        grid=(n // bn,),
        in_specs=[
            pl.BlockSpec((bn, d), lambda i: (i, 0)),
            pl.BlockSpec((bn, d), lambda i: (i, 0)),
            pl.BlockSpec((d, d), lambda i: (0, 0)),
            pl.BlockSpec((1, d), lambda i: (0, 0)),
            pl.BlockSpec((1, d), lambda i: (0, 0)),
            pl.BlockSpec((1, d), lambda i: (0, 0)),
        ],
        out_specs=[
            pl.BlockSpec((bn, d), lambda i: (i, 0)),
            pl.BlockSpec((bn, d), lambda i: (i, 0)),
        ],
        out_shape=[
            jax.ShapeDtypeStruct((n, d), _F32),
            jax.ShapeDtypeStruct((n, d), _F32),
        ],
    )(ctx, x, w, b.reshape(1, d), g.reshape(1, d), bb.reshape(1, d))


def _ffn_kern(y_ref, xr_ref, w1_ref, b1_ref, w2_ref, b2_ref, g_ref, bb_ref, o_ref):
    h = _bdot(y_ref[...], w1_ref[...]) + b1_ref[...]
    h = 0.5 * h * (1.0 + jax.lax.erf(h * (1.0 / math.sqrt(2.0))))
    y2 = _bdot(h, w2_ref[...]) + b2_ref[...]
    o_ref[...] = _ln(xr_ref[...] + y2, g_ref[...], bb_ref[...])


def _ffn(y, xres, w1, b1, w2, b2, g, bb, bn=256):
    n, d = y.shape
    dff = w1.shape[1]
    return pl.pallas_call(
        _ffn_kern,
        grid=(n // bn,),
        in_specs=[
            pl.BlockSpec((bn, d), lambda i: (i, 0)),
            pl.BlockSpec((bn, d), lambda i: (i, 0)),
            pl.BlockSpec((d, dff), lambda i: (0, 0)),
            pl.BlockSpec((1, dff), lambda i: (0, 0)),
            pl.BlockSpec((dff, d), lambda i: (0, 0)),
            pl.BlockSpec((1, d), lambda i: (0, 0)),
            pl.BlockSpec((1, d), lambda i: (0, 0)),
            pl.BlockSpec((1, d), lambda i: (0, 0)),
        ],
        out_specs=pl.BlockSpec((bn, d), lambda i: (i, 0)),
        out_shape=jax.ShapeDtypeStruct((n, d), _F32),
    )(
        y,
        xres,
        w1,
        b1.reshape(1, dff),
        w2,
        b2.reshape(1, d),
        g.reshape(1, d),
        bb.reshape(1, d),
    )


def _distill_kern(x_ref, w_ref, b_ref, g_ref, bb_ref, o_ref, *, bl, nblk):
    j = pl.program_id(1)
    d = x_ref.shape[2]
    xb = x_ref[0, pl.ds(j * bl, bl + 4), :]
    y = (
        _bdot(xb[: bl + 2], w_ref[0])
        + _bdot(xb[1 : bl + 3], w_ref[1])
        + _bdot(xb[2 : bl + 4], w_ref[2])
        + b_ref[...]
    )
    y = y * (1.0 / math.sqrt(1.0 + 1e-5)) * g_ref[...] + bb_ref[...]
    y = jnp.where(y > 0.0, y, jnp.exp(y) - 1.0)
    ri = jax.lax.broadcasted_iota(jnp.int32, (bl + 2, 1), 0)
    edge = ((ri == 0) & (j == 0)) | ((ri == bl + 1) & (j == nblk - 1))
    y = jnp.where(edge, -jnp.inf, y)
    o_ref[0] = jnp.maximum(jnp.maximum(y[:bl], y[1 : bl + 1]), y[2 : bl + 2])


def _distill(x, w, b, g, bb, bl=256):
    bsz, l, d = x.shape
    xext = jnp.concatenate(
        [x[:, -2:, :], x, x[:, :2, :], jnp.zeros((bsz, 4, d), _F32)], axis=1
    )
    nblk = l // bl
    kern = functools.partial(_distill_kern, bl=bl, nblk=nblk)
    z = pl.pallas_call(
        kern,
        grid=(bsz, nblk),
        in_specs=[
            pl.BlockSpec((1, l + 8, d), lambda i, j: (i, 0, 0)),
            pl.BlockSpec((3, d, d), lambda i, j: (0, 0, 0)),
            pl.BlockSpec((1, d), lambda i, j: (0, 0)),
            pl.BlockSpec((1, d), lambda i, j: (0, 0)),
            pl.BlockSpec((1, d), lambda i, j: (0, 0)),
        ],
        out_specs=pl.BlockSpec((1, bl, d), lambda i, j: (i, j, 0)),
        out_shape=jax.ShapeDtypeStruct((bsz, l, d), _F32),
    )(xext, w, b.reshape(1, d), g.reshape(1, d), bb.reshape(1, d))
    return z[:, ::2, :]


def _head_kern(h_ref, x_ref, g_ref, b_ref, hw_ref, hb_ref, sw_ref, sb_ref, o_ref):
    for bi in range(B):
        h = _ln(h_ref[bi], g_ref[...], b_ref[...])
        pooled = jnp.mean(h, axis=0, keepdims=True)
        xm = jnp.mean(x_ref[bi], axis=0, keepdims=True)
        o_ref[pl.ds(bi, 1), :] = (
            jnp.dot(pooled, hw_ref[...], preferred_element_type=_F32)
            + hb_ref[...]
            + jnp.dot(xm, sw_ref[...], preferred_element_type=_F32)
            + sb_ref[...]
        )


def _head(h, x, g, b, hw, hb, sw, sb):
    _, l2, d = h.shape
    return pl.pallas_call(
        _head_kern,
        out_shape=jax.ShapeDtypeStruct((B, PRED_LEN), _F32),
    )(
        h,
        x,
        g.reshape(1, d),
        b.reshape(1, d),
        hw,
        hb.reshape(1, PRED_LEN),
        sw,
        sb.reshape(1, PRED_LEN),
    )


def _cnt_kern(idx_ref, c_ref, *, ns):
    bq, lq = c_ref.shape
    ji = jax.lax.broadcasted_iota(jnp.int32, (bq, lq), 1)
    idxb = idx_ref[...]
    c = jnp.zeros((bq, lq), _F32)
    for s in range(ns):
        c += (idxb[:, s : s + 1] == ji).astype(_F32)
    c_ref[...] = c


def _cnt_build(idx, lq, bq=256):
    ns = idx.shape[1]
    return pl.pallas_call(
        functools.partial(_cnt_kern, ns=ns),
        grid=(lq // bq,),
        in_specs=[pl.BlockSpec((bq, ns), lambda i: (i, 0))],
        out_specs=pl.BlockSpec((bq, lq), lambda i: (i, 0)),
        out_shape=jax.ShapeDtypeStruct((lq, lq), _F32),
    )(idx)


def _pos_emb(l, d):
    position = jnp.arange(l, dtype=_F32)[:, None]
    div = jnp.exp(
        jnp.arange(0, d, 2, dtype=_F32) * (-math.log(10000.0) / d)
    )
    ang = position * div
    return jnp.stack([jnp.sin(ang), jnp.cos(ang)], axis=-1).reshape(l, d)


def _sample_consts(layer_i, lq):
    sample_k = min(lq, int(FACTOR * math.log(lq + 1)))
    n_top = min(lq, int(FACTOR * math.log(lq + 1)))
    key = jax.random.fold_in(jax.random.key(42), layer_i)
    idx = jax.random.randint(key, (lq, sample_k), 0, lq)
    cnt = _cnt_build(idx, lq)
    return cnt, n_top


def kernel(x, params):
    p = params
    bsz, l, _ = x.shape

    xcat = jnp.concatenate(
        [jnp.roll(x, 1, axis=1), x, jnp.roll(x, -1, axis=1)], axis=-1
    ).reshape(bsz * l, 3 * C_IN)
    wcat = p["token_conv_w"].reshape(3 * C_IN, D_MODEL)
    h = _embed(xcat, wcat, _pos_emb(l, D_MODEL))

    lq = l
    for i in range(E_LAYERS):
        lp = p["layers"][i]
        cnt, n_top = _sample_consts(i, lq)
        b3 = jnp.stack([lp["q_b"], lp["k_b"], lp["v_b"]], axis=0)
        q, k, v = _qkv(h, lp["q_w"], lp["k_w"], lp["v_w"], b3)
        ctx = _attention(
            q.reshape(bsz, lq, D_MODEL),
            k.reshape(bsz, lq, D_MODEL),
            v.reshape(bsz, lq, D_MODEL),
            cnt,
            n_top,
        )
        ctx = ctx.reshape(bsz * lq, D_MODEL)
        xres, y = _ores(ctx, h, lp["o_w"], lp["o_b"], lp["n1_g"], lp["n1_b"])
        h = _ffn(
            y, xres, lp["ff1_w"], lp["ff1_b"], lp["ff2_w"], lp["ff2_b"],
            lp["n2_g"], lp["n2_b"],
        )
        if i < E_LAYERS - 1:
            cp = p["convs"][i]
            h3 = h.reshape(bsz, lq, D_MODEL)
            h3 = _distill(h3, cp["conv_w"], cp["conv_b"], cp["bn_g"], cp["bn_b"])
            lq = lq // 2
            h = h3.reshape(bsz * lq, D_MODEL)

    h3 = h.reshape(bsz, lq, D_MODEL)
    return _head(
        h3, x, p["norm_g"], p["norm_b"], p["head_w"], p["head_b"],
        p["skip_w"], p["skip_b"],
    )


# speedup vs baseline: 1.7298x; 1.0194x over previous
import functools
import math

import jax
import jax.numpy as jnp
from jax.experimental import pallas as pl
from jax.experimental.pallas import tpu as pltpu

B, L0, C_IN = 2, 2048, 64
D_MODEL, N_HEADS, E_LAYERS, D_FF = 1024, 16, 3, 4096
FACTOR, PRED_LEN = 5, 96
DH = D_MODEL // N_HEADS

_F32 = jnp.float32


def _ln(x, g, b, eps=1e-5):
    m = jnp.mean(x, axis=-1, keepdims=True)
    v = jnp.mean((x - m) ** 2, axis=-1, keepdims=True)
    return (x - m) * jax.lax.rsqrt(v + eps) * g + b


def _embed_kern(xc_ref, w_ref, o_ref, *, bn, nb_l):
    d = w_ref.shape[1]
    row0 = (pl.program_id(0) % nb_l) * bn
    t = (row0 + jax.lax.broadcasted_iota(jnp.int32, (bn, d), 0)).astype(_F32)
    j = jax.lax.broadcasted_iota(jnp.int32, (bn, d), 1)
    odd = (j % 2).astype(_F32)
    div = jnp.exp((j - (j % 2)).astype(_F32) * (-math.log(10000.0) / d))
    pe = jnp.sin(t * div + odd * (0.5 * math.pi))
    o_ref[...] = (
        jnp.dot(xc_ref[...], w_ref[...], preferred_element_type=_F32) + pe
    )


def _embed(xcat, wcat, bn=256):
    n, din = xcat.shape
    d = wcat.shape[1]
    nb_l = L0 // bn
    return pl.pallas_call(
        functools.partial(_embed_kern, bn=bn, nb_l=nb_l),
        grid=(n // bn,),
        in_specs=[
            pl.BlockSpec((bn, din), lambda i: (i, 0)),
            pl.BlockSpec((din, d), lambda i: (0, 0)),
        ],
        out_specs=pl.BlockSpec((bn, d), lambda i: (i, 0)),
        out_shape=jax.ShapeDtypeStruct((n, d), _F32),
    )(xcat, wcat)


def _bdot(a, b):
    return jnp.dot(a, b, preferred_element_type=_F32)


def _linear_kern(x_ref, w_ref, b_ref, o_ref):
    o_ref[...] = _bdot(x_ref[...], w_ref[...]) + b_ref[...]


def _ntdot(a, b):
    return jax.lax.dot_general(
        a, b, (((1,), (1,)), ((), ())), preferred_element_type=_F32
    )


def _qkv_kern(x_ref, wq_ref, wk_ref, wv_ref, b_ref, q_ref, k_ref, v_ref):
    x = x_ref[...]
    d = wq_ref.shape[1]
    q_ref[...] = _bdot(x, wq_ref[...]) + b_ref[0:1, :]
    k_ref[...] = _bdot(x, wk_ref[...]) + b_ref[1:2, :]
    v_ref[...] = _bdot(x, wv_ref[...]) + b_ref[2:3, :]


def _qkv(x, wq, wk, wv, b3, bn=256):
    n, d = x.shape
    out = jax.ShapeDtypeStruct((n, d), _F32)
    return pl.pallas_call(
        _qkv_kern,
        grid=(n // bn,),
        in_specs=[
            pl.BlockSpec((bn, d), lambda i: (i, 0)),
            pl.BlockSpec((d, d), lambda i: (0, 0)),
            pl.BlockSpec((d, d), lambda i: (0, 0)),
            pl.BlockSpec((d, d), lambda i: (0, 0)),
            pl.BlockSpec((3, d), lambda i: (0, 0)),
        ],
        out_specs=[
            pl.BlockSpec((bn, d), lambda i: (i, 0)),
            pl.BlockSpec((bn, d), lambda i: (i, 0)),
            pl.BlockSpec((bn, d), lambda i: (i, 0)),
        ],
        out_shape=[out, out, out],
    )(x, wq, wk, wv, b3)


def _linear(x, w, b, bn=256):
    n, din = x.shape
    d = w.shape[1]
    return pl.pallas_call(
        _linear_kern,
        grid=(n // bn,),
        in_specs=[
            pl.BlockSpec((bn, din), lambda i: (i, 0)),
            pl.BlockSpec((din, d), lambda i: (0, 0)),
            pl.BlockSpec((1, d), lambda i: (0, 0)),
        ],
        out_specs=pl.BlockSpec((bn, d), lambda i: (i, 0)),
        out_shape=jax.ShapeDtypeStruct((n, d), _F32),
    )(x, w, b.reshape(1, d))


def _measure_kern(q_ref, k_ref, cnt_ref, m_ref, *, ns, bq):
    lq = q_ref.shape[1]
    for hh in range(2):
        c0, c1 = hh * DH, (hh + 1) * DH
        kh = k_ref[0, :, c0:c1]

        def mblk(i, carry):
            qb = q_ref[0, pl.ds(i * bq, bq), c0:c1]
            cb = cnt_ref[pl.ds(i * bq, bq), :]
            s = _ntdot(qb, kh)
            mean = jnp.sum(s * cb, axis=1) / ns
            mx = jnp.max(jnp.where(cb > 0.0, s, -jnp.inf), axis=1)
            m_ref[hh, 0, pl.ds(i * bq, bq)] = mx - mean
            return carry

        jax.lax.fori_loop(0, lq // bq, mblk, 0)


def _topk_kern(m_ref, t_ref, *, ns, nsp):
    bh, _, lq = m_ref.shape
    m = m_ref[:, 0, :]
    ji = jax.lax.broadcasted_iota(jnp.int32, (bh, lq), 1)
    ci = jax.lax.broadcasted_iota(jnp.int32, (bh, nsp), 1)

    def sel(t, carry):
        m, tops = carry
        mval = jnp.max(m, axis=1, keepdims=True)
        it = jnp.min(jnp.where(m >= mval, ji, lq), axis=1, keepdims=True)
        tops = jnp.where(ci == t, it, tops)
        m = jnp.where(ji == it, -jnp.inf, m)
        return m, tops

    _, tops = jax.lax.fori_loop(
        0, ns, sel, (m, jnp.full((bh, nsp), lq, jnp.int32))
    )
    t_ref[:, 0, :] = tops


def _ctx_kern(q_ref, k_ref, v_ref, t_ref, o_ref, *, ns):
    lq = q_ref.shape[1]
    for hh in range(2):
        c0, c1 = hh * DH, (hh + 1) * DH
        kh = k_ref[0, :, c0:c1]
        vh = v_ref[0, :, c0:c1]
        rows = [q_ref[0, pl.ds(t_ref[hh, 0, n], 1), c0:c1] for n in range(ns)]
        qred = jnp.concatenate(rows, axis=0)
        scores = _ntdot(qred, kh) * (1.0 / math.sqrt(DH))
        smax = jnp.max(scores, axis=1, keepdims=True)
        e = jnp.exp(scores - smax)
        attn = e / jnp.sum(e, axis=1, keepdims=True)
        ctx = jnp.dot(attn, vh, preferred_element_type=_F32)
        o_ref[0, :, c0:c1] = jnp.broadcast_to(
            jnp.mean(vh, axis=0, keepdims=True), vh.shape
        )
        for n in range(ns):
            o_ref[0, pl.ds(t_ref[hh, 0, n], 1), c0:c1] = ctx[n : n + 1, :]


def _attention(q, k, v, cnt, ns, bq=256):
    bsz, lq, d = q.shape
    ng = N_HEADS // 2
    bh = bsz * N_HEADS
    nsp = 64
    m = pl.pallas_call(
        functools.partial(_measure_kern, ns=ns, bq=min(bq, lq)),
        grid=(bsz, ng),
        in_specs=[
            pl.BlockSpec((1, lq, 2 * DH), lambda b, g: (b, 0, g)),
            pl.BlockSpec((1, lq, 2 * DH), lambda b, g: (b, 0, g)),
            pl.BlockSpec((lq, lq), lambda b, g: (0, 0)),
        ],
        out_specs=pl.BlockSpec((2, 1, lq), lambda b, g: (b * ng + g, 0, 0)),
        out_shape=jax.ShapeDtypeStruct((bh, 1, lq), _F32),
    )(q, k, cnt)
    tops = pl.pallas_call(
        functools.partial(_topk_kern, ns=ns, nsp=nsp),
        out_shape=jax.ShapeDtypeStruct((bh, 1, nsp), jnp.int32),
    )(m)
    return pl.pallas_call(
        functools.partial(_ctx_kern, ns=ns),
        grid=(bsz, ng),
        in_specs=[
            pl.BlockSpec((1, lq, 2 * DH), lambda b, g: (b, 0, g)),
            pl.BlockSpec((1, lq, 2 * DH), lambda b, g: (b, 0, g)),
            pl.BlockSpec((1, lq, 2 * DH), lambda b, g: (b, 0, g)),
            pl.BlockSpec(
                (2, 1, nsp), lambda b, g: (b * ng + g, 0, 0), memory_space=pltpu.SMEM
            ),
        ],
        out_specs=pl.BlockSpec((1, lq, 2 * DH), lambda b, g: (b, 0, g)),
        out_shape=jax.ShapeDtypeStruct((bsz, lq, d), _F32),
    )(q, k, v, tops)


def _ores_kern(ctx_ref, x_ref, w_ref, b_ref, g_ref, bb_ref, xres_ref, y_ref):
    xr = x_ref[...] + _bdot(ctx_ref[...], w_ref[...]) + b_ref[...]
    xres_ref[...] = xr
    y_ref[...] = _ln(xr, g_ref[...], bb_ref[...])


def _ores(ctx, x, w, b, g, bb, bn=256):
    n, d = x.shape
    return pl.pallas_call(
        _ores_kern,
        grid=(n // bn,),
        in_specs=[
            pl.BlockSpec((bn, d), lambda i: (i, 0)),
            pl.BlockSpec((bn, d), lambda i: (i, 0)),
            pl.BlockSpec((d, d), lambda i: (0, 0)),
            pl.BlockSpec((1, d), lambda i: (0, 0)),
            pl.BlockSpec((1, d), lambda i: (0, 0)),
            pl.BlockSpec((1, d), lambda i: (0, 0)),
        ],
        out_specs=[
            pl.BlockSpec((bn, d), lambda i: (i, 0)),
            pl.BlockSpec((bn, d), lambda i: (i, 0)),
        ],
        out_shape=[
            jax.ShapeDtypeStruct((n, d), _F32),
            jax.ShapeDtypeStruct((n, d), _F32),
        ],
    )(ctx, x, w, b.reshape(1, d), g.reshape(1, d), bb.reshape(1, d))


def _ffn_kern(y_ref, xr_ref, w1_ref, b1_ref, w2_ref, b2_ref, g_ref, bb_ref, o_ref):
    h = _bdot(y_ref[...], w1_ref[...]) + b1_ref[...]
    h = 0.5 * h * (1.0 + jax.lax.erf(h * (1.0 / math.sqrt(2.0))))
    y2 = _bdot(h, w2_ref[...]) + b2_ref[...]
    o_ref[...] = _ln(xr_ref[...] + y2, g_ref[...], bb_ref[...])


def _ffn(y, xres, w1, b1, w2, b2, g, bb, bn=256):
    n, d = y.shape
    dff = w1.shape[1]
    return pl.pallas_call(
        _ffn_kern,
        grid=(n // bn,),
        in_specs=[
            pl.BlockSpec((bn, d), lambda i: (i, 0)),
            pl.BlockSpec((bn, d), lambda i: (i, 0)),
            pl.BlockSpec((d, dff), lambda i: (0, 0)),
            pl.BlockSpec((1, dff), lambda i: (0, 0)),
            pl.BlockSpec((dff, d), lambda i: (0, 0)),
            pl.BlockSpec((1, d), lambda i: (0, 0)),
            pl.BlockSpec((1, d), lambda i: (0, 0)),
            pl.BlockSpec((1, d), lambda i: (0, 0)),
        ],
        out_specs=pl.BlockSpec((bn, d), lambda i: (i, 0)),
        out_shape=jax.ShapeDtypeStruct((n, d), _F32),
    )(
        y,
        xres,
        w1,
        b1.reshape(1, dff),
        w2,
        b2.reshape(1, d),
        g.reshape(1, d),
        bb.reshape(1, d),
    )


def _distill_kern(x_ref, w_ref, b_ref, g_ref, bb_ref, o_ref, *, bl, nblk):
    j = pl.program_id(1)
    d = x_ref.shape[2]
    xb = x_ref[0, pl.ds(j * bl, bl + 4), :]
    y = (
        _bdot(xb[: bl + 2], w_ref[0])
        + _bdot(xb[1 : bl + 3], w_ref[1])
        + _bdot(xb[2 : bl + 4], w_ref[2])
        + b_ref[...]
    )
    y = y * (1.0 / math.sqrt(1.0 + 1e-5)) * g_ref[...] + bb_ref[...]
    y = jnp.where(y > 0.0, y, jnp.exp(y) - 1.0)
    ri = jax.lax.broadcasted_iota(jnp.int32, (bl + 2, 1), 0)
    edge = ((ri == 0) & (j == 0)) | ((ri == bl + 1) & (j == nblk - 1))
    y = jnp.where(edge, -jnp.inf, y)
    o_ref[0] = jnp.maximum(jnp.maximum(y[:bl], y[1 : bl + 1]), y[2 : bl + 2])


def _distill(x, w, b, g, bb, bl=256):
    bsz, l, d = x.shape
    xext = jnp.concatenate(
        [x[:, -2:, :], x, x[:, :2, :], jnp.zeros((bsz, 4, d), _F32)], axis=1
    )
    nblk = l // bl
    kern = functools.partial(_distill_kern, bl=bl, nblk=nblk)
    z = pl.pallas_call(
        kern,
        grid=(bsz, nblk),
        in_specs=[
            pl.BlockSpec((1, l + 8, d), lambda i, j: (i, 0, 0)),
            pl.BlockSpec((3, d, d), lambda i, j: (0, 0, 0)),
            pl.BlockSpec((1, d), lambda i, j: (0, 0)),
            pl.BlockSpec((1, d), lambda i, j: (0, 0)),
            pl.BlockSpec((1, d), lambda i, j: (0, 0)),
        ],
        out_specs=pl.BlockSpec((1, bl, d), lambda i, j: (i, j, 0)),
        out_shape=jax.ShapeDtypeStruct((bsz, l, d), _F32),
    )(xext, w, b.reshape(1, d), g.reshape(1, d), bb.reshape(1, d))
    return z[:, ::2, :]


def _head_kern(h_ref, x_ref, g_ref, b_ref, hw_ref, hb_ref, sw_ref, sb_ref, o_ref):
    for bi in range(B):
        h = _ln(h_ref[bi], g_ref[...], b_ref[...])
        pooled = jnp.mean(h, axis=0, keepdims=True)
        xm = jnp.mean(x_ref[bi], axis=0, keepdims=True)
        o_ref[pl.ds(bi, 1), :] = (
            jnp.dot(pooled, hw_ref[...], preferred_element_type=_F32)
            + hb_ref[...]
            + jnp.dot(xm, sw_ref[...], preferred_element_type=_F32)
            + sb_ref[...]
        )


def _head(h, x, g, b, hw, hb, sw, sb):
    _, l2, d = h.shape
    return pl.pallas_call(
        _head_kern,
        out_shape=jax.ShapeDtypeStruct((B, PRED_LEN), _F32),
    )(
        h,
        x,
        g.reshape(1, d),
        b.reshape(1, d),
        hw,
        hb.reshape(1, PRED_LEN),
        sw,
        sb.reshape(1, PRED_LEN),
    )


def _cnt_kern(idx_ref, c_ref, *, ns):
    bq, lq = c_ref.shape
    ji = jax.lax.broadcasted_iota(jnp.int32, (bq, lq), 1)
    idxb = idx_ref[...]
    c = jnp.zeros((bq, lq), _F32)
    for s in range(ns):
        c += (idxb[:, s : s + 1] == ji).astype(_F32)
    c_ref[...] = c


def _cnt_build(idx, lq, bq=256):
    ns = idx.shape[1]
    return pl.pallas_call(
        functools.partial(_cnt_kern, ns=ns),
        grid=(lq // bq,),
        in_specs=[pl.BlockSpec((bq, ns), lambda i: (i, 0))],
        out_specs=pl.BlockSpec((bq, lq), lambda i: (i, 0)),
        out_shape=jax.ShapeDtypeStruct((lq, lq), _F32),
    )(idx)


def _pos_emb(l, d):
    position = jnp.arange(l, dtype=_F32)[:, None]
    div = jnp.exp(
        jnp.arange(0, d, 2, dtype=_F32) * (-math.log(10000.0) / d)
    )
    ang = position * div
    return jnp.stack([jnp.sin(ang), jnp.cos(ang)], axis=-1).reshape(l, d)


def _sample_consts(layer_i, lq):
    sample_k = min(lq, int(FACTOR * math.log(lq + 1)))
    n_top = min(lq, int(FACTOR * math.log(lq + 1)))
    key = jax.random.fold_in(jax.random.key(42), layer_i)
    idx = jax.random.randint(key, (lq, sample_k), 0, lq)
    cnt = _cnt_build(idx, lq)
    return cnt, n_top


def kernel(x, params):
    p = params
    bsz, l, _ = x.shape

    xcat = jnp.concatenate(
        [jnp.roll(x, 1, axis=1), x, jnp.roll(x, -1, axis=1)], axis=-1
    ).reshape(bsz * l, 3 * C_IN)
    wcat = p["token_conv_w"].reshape(3 * C_IN, D_MODEL)
    h = _embed(xcat, wcat)

    lq = l
    for i in range(E_LAYERS):
        lp = p["layers"][i]
        cnt, n_top = _sample_consts(i, lq)
        b3 = jnp.stack([lp["q_b"], lp["k_b"], lp["v_b"]], axis=0)
        q, k, v = _qkv(h, lp["q_w"], lp["k_w"], lp["v_w"], b3)
        ctx = _attention(
            q.reshape(bsz, lq, D_MODEL),
            k.reshape(bsz, lq, D_MODEL),
            v.reshape(bsz, lq, D_MODEL),
            cnt,
            n_top,
        )
        ctx = ctx.reshape(bsz * lq, D_MODEL)
        xres, y = _ores(ctx, h, lp["o_w"], lp["o_b"], lp["n1_g"], lp["n1_b"])
        h = _ffn(
            y, xres, lp["ff1_w"], lp["ff1_b"], lp["ff2_w"], lp["ff2_b"],
            lp["n2_g"], lp["n2_b"],
        )
        if i < E_LAYERS - 1:
            cp = p["convs"][i]
            h3 = h.reshape(bsz, lq, D_MODEL)
            h3 = _distill(h3, cp["conv_w"], cp["conv_b"], cp["bn_g"], cp["bn_b"])
            lq = lq // 2
            h = h3.reshape(bsz * lq, D_MODEL)

    h3 = h.reshape(bsz, lq, D_MODEL)
    return _head(
        h3, x, p["norm_g"], p["norm_b"], p["head_w"], p["head_b"],
        p["skip_w"], p["skip_b"],
    )


# speedup vs baseline: 1.7845x; 1.0316x over previous
import functools
import math

import jax
import jax.numpy as jnp
from jax.experimental import pallas as pl
from jax.experimental.pallas import tpu as pltpu

B, L0, C_IN = 2, 2048, 64
D_MODEL, N_HEADS, E_LAYERS, D_FF = 1024, 16, 3, 4096
FACTOR, PRED_LEN = 5, 96
DH = D_MODEL // N_HEADS

_F32 = jnp.float32


def _ln(x, g, b, eps=1e-5):
    m = jnp.mean(x, axis=-1, keepdims=True)
    v = jnp.mean((x - m) ** 2, axis=-1, keepdims=True)
    return (x - m) * jax.lax.rsqrt(v + eps) * g + b


def _embed_kern(xc_ref, w_ref, o_ref, *, bn, nb_l):
    d = w_ref.shape[1]
    row0 = (pl.program_id(0) % nb_l) * bn
    t = (row0 + jax.lax.broadcasted_iota(jnp.int32, (bn, d), 0)).astype(_F32)
    j = jax.lax.broadcasted_iota(jnp.int32, (bn, d), 1)
    odd = (j % 2).astype(_F32)
    div = jnp.exp((j - (j % 2)).astype(_F32) * (-math.log(10000.0) / d))
    pe = jnp.sin(t * div + odd * (0.5 * math.pi))
    o_ref[...] = (
        jnp.dot(xc_ref[...], w_ref[...], preferred_element_type=_F32) + pe
    )


def _embed(xcat, wcat, bn=256):
    n, din = xcat.shape
    d = wcat.shape[1]
    nb_l = L0 // bn
    return pl.pallas_call(
        functools.partial(_embed_kern, bn=bn, nb_l=nb_l),
        grid=(n // bn,),
        in_specs=[
            pl.BlockSpec((bn, din), lambda i: (i, 0)),
            pl.BlockSpec((din, d), lambda i: (0, 0)),
        ],
        out_specs=pl.BlockSpec((bn, d), lambda i: (i, 0)),
        out_shape=jax.ShapeDtypeStruct((n, d), _F32),
    )(xcat, wcat)


def _bdot(a, b):
    return jnp.dot(a, b, preferred_element_type=_F32)


def _linear_kern(x_ref, w_ref, b_ref, o_ref):
    o_ref[...] = _bdot(x_ref[...], w_ref[...]) + b_ref[...]


def _ntdot(a, b):
    return jax.lax.dot_general(
        a, b, (((1,), (1,)), ((), ())), preferred_element_type=_F32
    )


def _qkv_kern(x_ref, wq_ref, wk_ref, wv_ref, b_ref, q_ref, k_ref, v_ref):
    x = x_ref[...]
    d = wq_ref.shape[1]
    q_ref[...] = _bdot(x, wq_ref[...]) + b_ref[0:1, :]
    k_ref[...] = _bdot(x, wk_ref[...]) + b_ref[1:2, :]
    v_ref[...] = _bdot(x, wv_ref[...]) + b_ref[2:3, :]


def _qkv(x, wq, wk, wv, b3, bn=256):
    n, d = x.shape
    out = jax.ShapeDtypeStruct((n, d), _F32)
    return pl.pallas_call(
        _qkv_kern,
        grid=(n // bn,),
        in_specs=[
            pl.BlockSpec((bn, d), lambda i: (i, 0)),
            pl.BlockSpec((d, d), lambda i: (0, 0)),
            pl.BlockSpec((d, d), lambda i: (0, 0)),
            pl.BlockSpec((d, d), lambda i: (0, 0)),
            pl.BlockSpec((3, d), lambda i: (0, 0)),
        ],
        out_specs=[
            pl.BlockSpec((bn, d), lambda i: (i, 0)),
            pl.BlockSpec((bn, d), lambda i: (i, 0)),
            pl.BlockSpec((bn, d), lambda i: (i, 0)),
        ],
        out_shape=[out, out, out],
    )(x, wq, wk, wv, b3)


def _linear(x, w, b, bn=256):
    n, din = x.shape
    d = w.shape[1]
    return pl.pallas_call(
        _linear_kern,
        grid=(n // bn,),
        in_specs=[
            pl.BlockSpec((bn, din), lambda i: (i, 0)),
            pl.BlockSpec((din, d), lambda i: (0, 0)),
            pl.BlockSpec((1, d), lambda i: (0, 0)),
        ],
        out_specs=pl.BlockSpec((bn, d), lambda i: (i, 0)),
        out_shape=jax.ShapeDtypeStruct((n, d), _F32),
    )(x, w, b.reshape(1, d))


def _measure_kern(q_ref, k_ref, cnt_ref, m_ref, *, ns, bq):
    lq = q_ref.shape[1]
    for hh in range(2):
        c0, c1 = hh * DH, (hh + 1) * DH
        kh = k_ref[0, :, c0:c1]

        def mblk(i, carry):
            qb = q_ref[0, pl.ds(i * bq, bq), c0:c1]
            cb = cnt_ref[pl.ds(i * bq, bq), :]
            s = _ntdot(qb, kh)
            mean = jnp.sum(s * cb, axis=1) / ns
            mx = jnp.max(jnp.where(cb > 0.0, s, -jnp.inf), axis=1)
            m_ref[hh, 0, pl.ds(i * bq, bq)] = mx - mean
            return carry

        jax.lax.fori_loop(0, lq // bq, mblk, 0)


def _topk_kern(m_ref, t_ref, *, ns, nsp):
    bh, _, lq = m_ref.shape
    m = m_ref[:, 0, :]
    ji = jax.lax.broadcasted_iota(jnp.int32, (bh, lq), 1)
    ci = jax.lax.broadcasted_iota(jnp.int32, (bh, nsp), 1)

    def sel(t, carry):
        m, tops = carry
        mval = jnp.max(m, axis=1, keepdims=True)
        it = jnp.min(jnp.where(m >= mval, ji, lq), axis=1, keepdims=True)
        tops = jnp.where(ci == t, it, tops)
        m = jnp.where(ji == it, -jnp.inf, m)
        return m, tops

    _, tops = jax.lax.fori_loop(
        0, ns, sel, (m, jnp.full((bh, nsp), lq, jnp.int32))
    )
    t_ref[:, 0, :] = tops


def _ctx_kern(q_ref, k_ref, v_ref, t_ref, o_ref, *, ns):
    lq = q_ref.shape[1]
    for hh in range(2):
        c0, c1 = hh * DH, (hh + 1) * DH
        kh = k_ref[0, :, c0:c1]
        vh = v_ref[0, :, c0:c1]
        rows = [q_ref[0, pl.ds(t_ref[hh, 0, n], 1), c0:c1] for n in range(ns)]
        qred = jnp.concatenate(rows, axis=0)
        scores = _ntdot(qred, kh) * (1.0 / math.sqrt(DH))
        smax = jnp.max(scores, axis=1, keepdims=True)
        e = jnp.exp(scores - smax)
        attn = e / jnp.sum(e, axis=1, keepdims=True)
        ctx = jnp.dot(attn, vh, preferred_element_type=_F32)
        o_ref[0, :, c0:c1] = jnp.broadcast_to(
            jnp.mean(vh, axis=0, keepdims=True), vh.shape
        )
        for n in range(ns):
            o_ref[0, pl.ds(t_ref[hh, 0, n], 1), c0:c1] = ctx[n : n + 1, :]


def _attention(q, k, v, cnt, ns, bq=512):
    bsz, lq, d = q.shape
    ng = N_HEADS // 2
    bh = bsz * N_HEADS
    nsp = 64
    m = pl.pallas_call(
        functools.partial(_measure_kern, ns=ns, bq=min(bq, lq)),
        grid=(bsz, ng),
        in_specs=[
            pl.BlockSpec((1, lq, 2 * DH), lambda b, g: (b, 0, g)),
            pl.BlockSpec((1, lq, 2 * DH), lambda b, g: (b, 0, g)),
            pl.BlockSpec((lq, lq), lambda b, g: (0, 0)),
        ],
        out_specs=pl.BlockSpec((2, 1, lq), lambda b, g: (b * ng + g, 0, 0)),
        out_shape=jax.ShapeDtypeStruct((bh, 1, lq), _F32),
    )(q, k, cnt)
    tops = pl.pallas_call(
        functools.partial(_topk_kern, ns=ns, nsp=nsp),
        out_shape=jax.ShapeDtypeStruct((bh, 1, nsp), jnp.int32),
    )(m)
    return pl.pallas_call(
        functools.partial(_ctx_kern, ns=ns),
        grid=(bsz, ng),
        in_specs=[
            pl.BlockSpec((1, lq, 2 * DH), lambda b, g: (b, 0, g)),
            pl.BlockSpec((1, lq, 2 * DH), lambda b, g: (b, 0, g)),
            pl.BlockSpec((1, lq, 2 * DH), lambda b, g: (b, 0, g)),
            pl.BlockSpec(
                (2, 1, nsp), lambda b, g: (b * ng + g, 0, 0), memory_space=pltpu.SMEM
            ),
        ],
        out_specs=pl.BlockSpec((1, lq, 2 * DH), lambda b, g: (b, 0, g)),
        out_shape=jax.ShapeDtypeStruct((bsz, lq, d), _F32),
    )(q, k, v, tops)


def _ores_kern(ctx_ref, x_ref, w_ref, b_ref, g_ref, bb_ref, xres_ref, y_ref):
    xr = x_ref[...] + _bdot(ctx_ref[...], w_ref[...]) + b_ref[...]
    xres_ref[...] = xr
    y_ref[...] = _ln(xr, g_ref[...], bb_ref[...])


def _ores(ctx, x, w, b, g, bb, bn=256):
    n, d = x.shape
    return pl.pallas_call(
        _ores_kern,
        grid=(n // bn,),
        in_specs=[
            pl.BlockSpec((bn, d), lambda i: (i, 0)),
            pl.BlockSpec((bn, d), lambda i: (i, 0)),
            pl.BlockSpec((d, d), lambda i: (0, 0)),
            pl.BlockSpec((1, d), lambda i: (0, 0)),
            pl.BlockSpec((1, d), lambda i: (0, 0)),
            pl.BlockSpec((1, d), lambda i: (0, 0)),
        ],
        out_specs=[
            pl.BlockSpec((bn, d), lambda i: (i, 0)),
            pl.BlockSpec((bn, d), lambda i: (i, 0)),
        ],
        out_shape=[
            jax.ShapeDtypeStruct((n, d), _F32),
            jax.ShapeDtypeStruct((n, d), _F32),
        ],
    )(ctx, x, w, b.reshape(1, d), g.reshape(1, d), bb.reshape(1, d))


def _ffn_kern(y_ref, xr_ref, w1_ref, b1_ref, w2_ref, b2_ref, g_ref, bb_ref, o_ref):
    h = _bdot(y_ref[...], w1_ref[...]) + b1_ref[...]
    h = 0.5 * h * (1.0 + jax.lax.erf(h * (1.0 / math.sqrt(2.0))))
    y2 = _bdot(h, w2_ref[...]) + b2_ref[...]
    o_ref[...] = _ln(xr_ref[...] + y2, g_ref[...], bb_ref[...])


def _ffn(y, xres, w1, b1, w2, b2, g, bb, bn=256):
    n, d = y.shape
    dff = w1.shape[1]
    return pl.pallas_call(
        _ffn_kern,
        grid=(n // bn,),
        in_specs=[
            pl.BlockSpec((bn, d), lambda i: (i, 0)),
            pl.BlockSpec((bn, d), lambda i: (i, 0)),
            pl.BlockSpec((d, dff), lambda i: (0, 0)),
            pl.BlockSpec((1, dff), lambda i: (0, 0)),
            pl.BlockSpec((dff, d), lambda i: (0, 0)),
            pl.BlockSpec((1, d), lambda i: (0, 0)),
            pl.BlockSpec((1, d), lambda i: (0, 0)),
            pl.BlockSpec((1, d), lambda i: (0, 0)),
        ],
        out_specs=pl.BlockSpec((bn, d), lambda i: (i, 0)),
        out_shape=jax.ShapeDtypeStruct((n, d), _F32),
    )(
        y,
        xres,
        w1,
        b1.reshape(1, dff),
        w2,
        b2.reshape(1, d),
        g.reshape(1, d),
        bb.reshape(1, d),
    )


def _distill_kern(x_ref, w_ref, b_ref, g_ref, bb_ref, o_ref, *, bl, nblk):
    j = pl.program_id(1)
    d = x_ref.shape[2]
    xb = x_ref[0, pl.ds(j * bl, bl + 4), :]
    y = (
        _bdot(xb[: bl + 2], w_ref[0])
        + _bdot(xb[1 : bl + 3], w_ref[1])
        + _bdot(xb[2 : bl + 4], w_ref[2])
        + b_ref[...]
    )
    y = y * (1.0 / math.sqrt(1.0 + 1e-5)) * g_ref[...] + bb_ref[...]
    y = jnp.where(y > 0.0, y, jnp.exp(y) - 1.0)
    ri = jax.lax.broadcasted_iota(jnp.int32, (bl + 2, 1), 0)
    edge = ((ri == 0) & (j == 0)) | ((ri == bl + 1) & (j == nblk - 1))
    y = jnp.where(edge, -jnp.inf, y)
    o_ref[0] = jnp.maximum(jnp.maximum(y[:bl], y[1 : bl + 1]), y[2 : bl + 2])


def _distill(x, w, b, g, bb, bl=256):
    bsz, l, d = x.shape
    xext = jnp.concatenate(
        [x[:, -2:, :], x, x[:, :2, :], jnp.zeros((bsz, 4, d), _F32)], axis=1
    )
    nblk = l // bl
    kern = functools.partial(_distill_kern, bl=bl, nblk=nblk)
    z = pl.pallas_call(
        kern,
        grid=(bsz, nblk),
        in_specs=[
            pl.BlockSpec((1, l + 8, d), lambda i, j: (i, 0, 0)),
            pl.BlockSpec((3, d, d), lambda i, j: (0, 0, 0)),
            pl.BlockSpec((1, d), lambda i, j: (0, 0)),
            pl.BlockSpec((1, d), lambda i, j: (0, 0)),
            pl.BlockSpec((1, d), lambda i, j: (0, 0)),
        ],
        out_specs=pl.BlockSpec((1, bl, d), lambda i, j: (i, j, 0)),
        out_shape=jax.ShapeDtypeStruct((bsz, l, d), _F32),
    )(xext, w, b.reshape(1, d), g.reshape(1, d), bb.reshape(1, d))
    return z[:, ::2, :]


def _head_kern(h_ref, x_ref, g_ref, b_ref, hw_ref, hb_ref, sw_ref, sb_ref, o_ref):
    for bi in range(B):
        h = _ln(h_ref[bi], g_ref[...], b_ref[...])
        pooled = jnp.mean(h, axis=0, keepdims=True)
        xm = jnp.mean(x_ref[bi], axis=0, keepdims=True)
        o_ref[pl.ds(bi, 1), :] = (
            jnp.dot(pooled, hw_ref[...], preferred_element_type=_F32)
            + hb_ref[...]
            + jnp.dot(xm, sw_ref[...], preferred_element_type=_F32)
            + sb_ref[...]
        )


def _head(h, x, g, b, hw, hb, sw, sb):
    _, l2, d = h.shape
    return pl.pallas_call(
        _head_kern,
        out_shape=jax.ShapeDtypeStruct((B, PRED_LEN), _F32),
    )(
        h,
        x,
        g.reshape(1, d),
        b.reshape(1, d),
        hw,
        hb.reshape(1, PRED_LEN),
        sw,
        sb.reshape(1, PRED_LEN),
    )


def _cnt_kern(idx_ref, c_ref, *, ns):
    bq, lq = c_ref.shape
    ji = jax.lax.broadcasted_iota(jnp.int32, (bq, lq), 1)
    idxb = idx_ref[...]
    c = jnp.zeros((bq, lq), _F32)
    for s in range(ns):
        c += (idxb[:, s : s + 1] == ji).astype(_F32)
    c_ref[...] = c


def _cnt_build(idx, lq, bq=256):
    ns = idx.shape[1]
    return pl.pallas_call(
        functools.partial(_cnt_kern, ns=ns),
        grid=(lq // bq,),
        in_specs=[pl.BlockSpec((bq, ns), lambda i: (i, 0))],
        out_specs=pl.BlockSpec((bq, lq), lambda i: (i, 0)),
        out_shape=jax.ShapeDtypeStruct((lq, lq), _F32),
    )(idx)


def _pos_emb(l, d):
    position = jnp.arange(l, dtype=_F32)[:, None]
    div = jnp.exp(
        jnp.arange(0, d, 2, dtype=_F32) * (-math.log(10000.0) / d)
    )
    ang = position * div
    return jnp.stack([jnp.sin(ang), jnp.cos(ang)], axis=-1).reshape(l, d)


def _sample_consts(layer_i, lq):
    sample_k = min(lq, int(FACTOR * math.log(lq + 1)))
    n_top = min(lq, int(FACTOR * math.log(lq + 1)))
    key = jax.random.fold_in(jax.random.key(42), layer_i)
    idx = jax.random.randint(key, (lq, sample_k), 0, lq)
    cnt = _cnt_build(idx, lq)
    return cnt, n_top


def kernel(x, params):
    p = params
    bsz, l, _ = x.shape

    xcat = jnp.concatenate(
        [jnp.roll(x, 1, axis=1), x, jnp.roll(x, -1, axis=1)], axis=-1
    ).reshape(bsz * l, 3 * C_IN)
    wcat = p["token_conv_w"].reshape(3 * C_IN, D_MODEL)
    h = _embed(xcat, wcat)

    lq = l
    for i in range(E_LAYERS):
        lp = p["layers"][i]
        cnt, n_top = _sample_consts(i, lq)
        b3 = jnp.stack([lp["q_b"], lp["k_b"], lp["v_b"]], axis=0)
        q, k, v = _qkv(h, lp["q_w"], lp["k_w"], lp["v_w"], b3)
        ctx = _attention(
            q.reshape(bsz, lq, D_MODEL),
            k.reshape(bsz, lq, D_MODEL),
            v.reshape(bsz, lq, D_MODEL),
            cnt,
            n_top,
        )
        ctx = ctx.reshape(bsz * lq, D_MODEL)
        xres, y = _ores(ctx, h, lp["o_w"], lp["o_b"], lp["n1_g"], lp["n1_b"])
        h = _ffn(
            y, xres, lp["ff1_w"], lp["ff1_b"], lp["ff2_w"], lp["ff2_b"],
            lp["n2_g"], lp["n2_b"],
        )
        if i < E_LAYERS - 1:
            cp = p["convs"][i]
            h3 = h.reshape(bsz, lq, D_MODEL)
            h3 = _distill(h3, cp["conv_w"], cp["conv_b"], cp["bn_g"], cp["bn_b"])
            lq = lq // 2
            h = h3.reshape(bsz * lq, D_MODEL)

    h3 = h.reshape(bsz, lq, D_MODEL)
    return _head(
        h3, x, p["norm_g"], p["norm_b"], p["head_w"], p["head_b"],
        p["skip_w"], p["skip_b"],
    )


# speedup vs baseline: 1.8145x; 1.0168x over previous
import functools
import math

import jax
import jax.numpy as jnp
from jax.experimental import pallas as pl
from jax.experimental.pallas import tpu as pltpu

B, L0, C_IN = 2, 2048, 64
D_MODEL, N_HEADS, E_LAYERS, D_FF = 1024, 16, 3, 4096
FACTOR, PRED_LEN = 5, 96
DH = D_MODEL // N_HEADS

_F32 = jnp.float32


def _ln(x, g, b, eps=1e-5):
    m = jnp.mean(x, axis=-1, keepdims=True)
    v = jnp.mean((x - m) ** 2, axis=-1, keepdims=True)
    return (x - m) * jax.lax.rsqrt(v + eps) * g + b


def _embed_kern(xc_ref, w_ref, o_ref, *, bn, nb_l):
    d = w_ref.shape[1]
    row0 = (pl.program_id(0) % nb_l) * bn
    t = (row0 + jax.lax.broadcasted_iota(jnp.int32, (bn, d), 0)).astype(_F32)
    j = jax.lax.broadcasted_iota(jnp.int32, (bn, d), 1)
    odd = (j % 2).astype(_F32)
    div = jnp.exp((j - (j % 2)).astype(_F32) * (-math.log(10000.0) / d))
    pe = jnp.sin(t * div + odd * (0.5 * math.pi))
    o_ref[...] = (
        jnp.dot(xc_ref[...], w_ref[...], preferred_element_type=_F32) + pe
    )


def _embed(xcat, wcat, bn=512):
    n, din = xcat.shape
    d = wcat.shape[1]
    nb_l = L0 // bn
    return pl.pallas_call(
        functools.partial(_embed_kern, bn=bn, nb_l=nb_l),
        grid=(n // bn,),
        in_specs=[
            pl.BlockSpec((bn, din), lambda i: (i, 0)),
            pl.BlockSpec((din, d), lambda i: (0, 0)),
        ],
        out_specs=pl.BlockSpec((bn, d), lambda i: (i, 0)),
        out_shape=jax.ShapeDtypeStruct((n, d), _F32),
    )(xcat, wcat)


def _bdot(a, b):
    return jnp.dot(a, b, preferred_element_type=_F32)


def _linear_kern(x_ref, w_ref, b_ref, o_ref):
    o_ref[...] = _bdot(x_ref[...], w_ref[...]) + b_ref[...]


def _ntdot(a, b):
    return jax.lax.dot_general(
        a, b, (((1,), (1,)), ((), ())), preferred_element_type=_F32
    )


def _qkv_kern(x_ref, wq_ref, wk_ref, wv_ref, b_ref, q_ref, k_ref, v_ref):
    x = x_ref[...]
    d = wq_ref.shape[1]
    q_ref[...] = _bdot(x, wq_ref[...]) + b_ref[0:1, :]
    k_ref[...] = _bdot(x, wk_ref[...]) + b_ref[1:2, :]
    v_ref[...] = _bdot(x, wv_ref[...]) + b_ref[2:3, :]


def _qkv(x, wq, wk, wv, b3, bn=512):
    n, d = x.shape
    out = jax.ShapeDtypeStruct((n, d), _F32)
    return pl.pallas_call(
        _qkv_kern,
        grid=(n // bn,),
        in_specs=[
            pl.BlockSpec((bn, d), lambda i: (i, 0)),
            pl.BlockSpec((d, d), lambda i: (0, 0)),
            pl.BlockSpec((d, d), lambda i: (0, 0)),
            pl.BlockSpec((d, d), lambda i: (0, 0)),
            pl.BlockSpec((3, d), lambda i: (0, 0)),
        ],
        out_specs=[
            pl.BlockSpec((bn, d), lambda i: (i, 0)),
            pl.BlockSpec((bn, d), lambda i: (i, 0)),
            pl.BlockSpec((bn, d), lambda i: (i, 0)),
        ],
        out_shape=[out, out, out],
    )(x, wq, wk, wv, b3)


def _linear(x, w, b, bn=256):
    n, din = x.shape
    d = w.shape[1]
    return pl.pallas_call(
        _linear_kern,
        grid=(n // bn,),
        in_specs=[
            pl.BlockSpec((bn, din), lambda i: (i, 0)),
            pl.BlockSpec((din, d), lambda i: (0, 0)),
            pl.BlockSpec((1, d), lambda i: (0, 0)),
        ],
        out_specs=pl.BlockSpec((bn, d), lambda i: (i, 0)),
        out_shape=jax.ShapeDtypeStruct((n, d), _F32),
    )(x, w, b.reshape(1, d))


def _measure_kern(q_ref, k_ref, cnt_ref, m_ref, *, ns, bq):
    lq = q_ref.shape[1]
    for hh in range(2):
        c0, c1 = hh * DH, (hh + 1) * DH
        kh = k_ref[0, :, c0:c1]

        def mblk(i, carry):
            qb = q_ref[0, pl.ds(i * bq, bq), c0:c1]
            cb = cnt_ref[pl.ds(i * bq, bq), :]
            s = _ntdot(qb, kh)
            mean = jnp.sum(s * cb, axis=1) / ns
            mx = jnp.max(jnp.where(cb > 0.0, s, -jnp.inf), axis=1)
            m_ref[hh, 0, pl.ds(i * bq, bq)] = mx - mean
            return carry

        jax.lax.fori_loop(0, lq // bq, mblk, 0)


def _topk_kern(m_ref, t_ref, *, ns, nsp):
    bh, _, lq = m_ref.shape
    m = m_ref[:, 0, :]
    ji = jax.lax.broadcasted_iota(jnp.int32, (bh, lq), 1)
    ci = jax.lax.broadcasted_iota(jnp.int32, (bh, nsp), 1)

    def sel(t, carry):
        m, tops = carry
        mval = jnp.max(m, axis=1, keepdims=True)
        it = jnp.min(jnp.where(m >= mval, ji, lq), axis=1, keepdims=True)
        tops = jnp.where(ci == t, it, tops)
        m = jnp.where(ji == it, -jnp.inf, m)
        return m, tops

    _, tops = jax.lax.fori_loop(
        0, ns, sel, (m, jnp.full((bh, nsp), lq, jnp.int32))
    )
    t_ref[:, 0, :] = tops


def _ctx_kern(q_ref, k_ref, v_ref, t_ref, o_ref, *, ns):
    lq = q_ref.shape[1]
    for hh in range(2):
        c0, c1 = hh * DH, (hh + 1) * DH
        kh = k_ref[0, :, c0:c1]
        vh = v_ref[0, :, c0:c1]
        rows = [q_ref[0, pl.ds(t_ref[hh, 0, n], 1), c0:c1] for n in range(ns)]
        qred = jnp.concatenate(rows, axis=0)
        scores = _ntdot(qred, kh) * (1.0 / math.sqrt(DH))
        smax = jnp.max(scores, axis=1, keepdims=True)
        e = jnp.exp(scores - smax)
        attn = e / jnp.sum(e, axis=1, keepdims=True)
        ctx = jnp.dot(attn, vh, preferred_element_type=_F32)
        o_ref[0, :, c0:c1] = jnp.broadcast_to(
            jnp.mean(vh, axis=0, keepdims=True), vh.shape
        )
        for n in range(ns):
            o_ref[0, pl.ds(t_ref[hh, 0, n], 1), c0:c1] = ctx[n : n + 1, :]


def _attention(q, k, v, cnt, ns, bq=512):
    bsz, lq, d = q.shape
    ng = N_HEADS // 2
    bh = bsz * N_HEADS
    nsp = 64
    m = pl.pallas_call(
        functools.partial(_measure_kern, ns=ns, bq=min(bq, lq)),
        grid=(bsz, ng),
        in_specs=[
            pl.BlockSpec((1, lq, 2 * DH), lambda b, g: (b, 0, g)),
            pl.BlockSpec((1, lq, 2 * DH), lambda b, g: (b, 0, g)),
            pl.BlockSpec((lq, lq), lambda b, g: (0, 0)),
        ],
        out_specs=pl.BlockSpec((2, 1, lq), lambda b, g: (b * ng + g, 0, 0)),
        out_shape=jax.ShapeDtypeStruct((bh, 1, lq), _F32),
    )(q, k, cnt)
    tops = pl.pallas_call(
        functools.partial(_topk_kern, ns=ns, nsp=nsp),
        out_shape=jax.ShapeDtypeStruct((bh, 1, nsp), jnp.int32),
    )(m)
    return pl.pallas_call(
        functools.partial(_ctx_kern, ns=ns),
        grid=(bsz, ng),
        in_specs=[
            pl.BlockSpec((1, lq, 2 * DH), lambda b, g: (b, 0, g)),
            pl.BlockSpec((1, lq, 2 * DH), lambda b, g: (b, 0, g)),
            pl.BlockSpec((1, lq, 2 * DH), lambda b, g: (b, 0, g)),
            pl.BlockSpec(
                (2, 1, nsp), lambda b, g: (b * ng + g, 0, 0), memory_space=pltpu.SMEM
            ),
        ],
        out_specs=pl.BlockSpec((1, lq, 2 * DH), lambda b, g: (b, 0, g)),
        out_shape=jax.ShapeDtypeStruct((bsz, lq, d), _F32),
    )(q, k, v, tops)


def _ores_kern(ctx_ref, x_ref, w_ref, b_ref, g_ref, bb_ref, xres_ref, y_ref):
    xr = x_ref[...] + _bdot(ctx_ref[...], w_ref[...]) + b_ref[...]
    xres_ref[...] = xr
    y_ref[...] = _ln(xr, g_ref[...], bb_ref[...])


def _ores(ctx, x, w, b, g, bb, bn=512):
    n, d = x.shape
    return pl.pallas_call(
        _ores_kern,
        grid=(n // bn,),
        in_specs=[
            pl.BlockSpec((bn, d), lambda i: (i, 0)),
            pl.BlockSpec((bn, d), lambda i: (i, 0)),
            pl.BlockSpec((d, d), lambda i: (0, 0)),
            pl.BlockSpec((1, d), lambda i: (0, 0)),
            pl.BlockSpec((1, d), lambda i: (0, 0)),
            pl.BlockSpec((1, d), lambda i: (0, 0)),
        ],
        out_specs=[
            pl.BlockSpec((bn, d), lambda i: (i, 0)),
            pl.BlockSpec((bn, d), lambda i: (i, 0)),
        ],
        out_shape=[
            jax.ShapeDtypeStruct((n, d), _F32),
            jax.ShapeDtypeStruct((n, d), _F32),
        ],
    )(ctx, x, w, b.reshape(1, d), g.reshape(1, d), bb.reshape(1, d))


def _ffn_kern(y_ref, xr_ref, w1_ref, b1_ref, w2_ref, b2_ref, g_ref, bb_ref, o_ref):
    h = _bdot(y_ref[...], w1_ref[...]) + b1_ref[...]
    h = 0.5 * h * (1.0 + jax.lax.erf(h * (1.0 / math.sqrt(2.0))))
    y2 = _bdot(h, w2_ref[...]) + b2_ref[...]
    o_ref[...] = _ln(xr_ref[...] + y2, g_ref[...], bb_ref[...])


def _ffn(y, xres, w1, b1, w2, b2, g, bb, bn=256):
    n, d = y.shape
    dff = w1.shape[1]
    return pl.pallas_call(
        _ffn_kern,
        grid=(n // bn,),
        in_specs=[
            pl.BlockSpec((bn, d), lambda i: (i, 0)),
            pl.BlockSpec((bn, d), lambda i: (i, 0)),
            pl.BlockSpec((d, dff), lambda i: (0, 0)),
            pl.BlockSpec((1, dff), lambda i: (0, 0)),
            pl.BlockSpec((dff, d), lambda i: (0, 0)),
            pl.BlockSpec((1, d), lambda i: (0, 0)),
            pl.BlockSpec((1, d), lambda i: (0, 0)),
            pl.BlockSpec((1, d), lambda i: (0, 0)),
        ],
        out_specs=pl.BlockSpec((bn, d), lambda i: (i, 0)),
        out_shape=jax.ShapeDtypeStruct((n, d), _F32),
    )(
        y,
        xres,
        w1,
        b1.reshape(1, dff),
        w2,
        b2.reshape(1, d),
        g.reshape(1, d),
        bb.reshape(1, d),
    )


def _distill_kern(x_ref, w_ref, b_ref, g_ref, bb_ref, o_ref, *, bl, nblk):
    j = pl.program_id(1)
    d = x_ref.shape[2]
    xb = x_ref[0, pl.ds(j * bl, bl + 4), :]
    y = (
        _bdot(xb[: bl + 2], w_ref[0])
        + _bdot(xb[1 : bl + 3], w_ref[1])
        + _bdot(xb[2 : bl + 4], w_ref[2])
        + b_ref[...]
    )
    y = y * (1.0 / math.sqrt(1.0 + 1e-5)) * g_ref[...] + bb_ref[...]
    y = jnp.where(y > 0.0, y, jnp.exp(y) - 1.0)
    ri = jax.lax.broadcasted_iota(jnp.int32, (bl + 2, 1), 0)
    edge = ((ri == 0) & (j == 0)) | ((ri == bl + 1) & (j == nblk - 1))
    y = jnp.where(edge, -jnp.inf, y)
    o_ref[0] = jnp.maximum(jnp.maximum(y[:bl], y[1 : bl + 1]), y[2 : bl + 2])


def _distill(x, w, b, g, bb, bl=512):
    bsz, l, d = x.shape
    xext = jnp.concatenate(
        [x[:, -2:, :], x, x[:, :2, :], jnp.zeros((bsz, 4, d), _F32)], axis=1
    )
    nblk = l // bl
    kern = functools.partial(_distill_kern, bl=bl, nblk=nblk)
    z = pl.pallas_call(
        kern,
        grid=(bsz, nblk),
        in_specs=[
            pl.BlockSpec((1, l + 8, d), lambda i, j: (i, 0, 0)),
            pl.BlockSpec((3, d, d), lambda i, j: (0, 0, 0)),
            pl.BlockSpec((1, d), lambda i, j: (0, 0)),
            pl.BlockSpec((1, d), lambda i, j: (0, 0)),
            pl.BlockSpec((1, d), lambda i, j: (0, 0)),
        ],
        out_specs=pl.BlockSpec((1, bl, d), lambda i, j: (i, j, 0)),
        out_shape=jax.ShapeDtypeStruct((bsz, l, d), _F32),
    )(xext, w, b.reshape(1, d), g.reshape(1, d), bb.reshape(1, d))
    return z[:, ::2, :]


def _head_kern(h_ref, x_ref, g_ref, b_ref, hw_ref, hb_ref, sw_ref, sb_ref, o_ref):
    for bi in range(B):
        h = _ln(h_ref[bi], g_ref[...], b_ref[...])
        pooled = jnp.mean(h, axis=0, keepdims=True)
        xm = jnp.mean(x_ref[bi], axis=0, keepdims=True)
        o_ref[pl.ds(bi, 1), :] = (
            jnp.dot(pooled, hw_ref[...], preferred_element_type=_F32)
            + hb_ref[...]
            + jnp.dot(xm, sw_ref[...], preferred_element_type=_F32)
            + sb_ref[...]
        )


def _head(h, x, g, b, hw, hb, sw, sb):
    _, l2, d = h.shape
    return pl.pallas_call(
        _head_kern,
        out_shape=jax.ShapeDtypeStruct((B, PRED_LEN), _F32),
    )(
        h,
        x,
        g.reshape(1, d),
        b.reshape(1, d),
        hw,
        hb.reshape(1, PRED_LEN),
        sw,
        sb.reshape(1, PRED_LEN),
    )


def _cnt_kern(idx_ref, c_ref, *, ns):
    bq, lq = c_ref.shape
    ji = jax.lax.broadcasted_iota(jnp.int32, (bq, lq), 1)
    idxb = idx_ref[...]
    c = jnp.zeros((bq, lq), _F32)
    for s in range(ns):
        c += (idxb[:, s : s + 1] == ji).astype(_F32)
    c_ref[...] = c


def _cnt_build(idx, lq, bq=256):
    ns = idx.shape[1]
    return pl.pallas_call(
        functools.partial(_cnt_kern, ns=ns),
        grid=(lq // bq,),
        in_specs=[pl.BlockSpec((bq, ns), lambda i: (i, 0))],
        out_specs=pl.BlockSpec((bq, lq), lambda i: (i, 0)),
        out_shape=jax.ShapeDtypeStruct((lq, lq), _F32),
    )(idx)


def _pos_emb(l, d):
    position = jnp.arange(l, dtype=_F32)[:, None]
    div = jnp.exp(
        jnp.arange(0, d, 2, dtype=_F32) * (-math.log(10000.0) / d)
    )
    ang = position * div
    return jnp.stack([jnp.sin(ang), jnp.cos(ang)], axis=-1).reshape(l, d)


def _sample_consts(layer_i, lq):
    sample_k = min(lq, int(FACTOR * math.log(lq + 1)))
    n_top = min(lq, int(FACTOR * math.log(lq + 1)))
    key = jax.random.fold_in(jax.random.key(42), layer_i)
    idx = jax.random.randint(key, (lq, sample_k), 0, lq)
    cnt = _cnt_build(idx, lq)
    return cnt, n_top


def kernel(x, params):
    p = params
    bsz, l, _ = x.shape

    xcat = jnp.concatenate(
        [jnp.roll(x, 1, axis=1), x, jnp.roll(x, -1, axis=1)], axis=-1
    ).reshape(bsz * l, 3 * C_IN)
    wcat = p["token_conv_w"].reshape(3 * C_IN, D_MODEL)
    h = _embed(xcat, wcat)

    lq = l
    for i in range(E_LAYERS):
        lp = p["layers"][i]
        cnt, n_top = _sample_consts(i, lq)
        b3 = jnp.stack([lp["q_b"], lp["k_b"], lp["v_b"]], axis=0)
        q, k, v = _qkv(h, lp["q_w"], lp["k_w"], lp["v_w"], b3)
        ctx = _attention(
            q.reshape(bsz, lq, D_MODEL),
            k.reshape(bsz, lq, D_MODEL),
            v.reshape(bsz, lq, D_MODEL),
            cnt,
            n_top,
        )
        ctx = ctx.reshape(bsz * lq, D_MODEL)
        xres, y = _ores(ctx, h, lp["o_w"], lp["o_b"], lp["n1_g"], lp["n1_b"])
        h = _ffn(
            y, xres, lp["ff1_w"], lp["ff1_b"], lp["ff2_w"], lp["ff2_b"],
            lp["n2_g"], lp["n2_b"],
        )
        if i < E_LAYERS - 1:
            cp = p["convs"][i]
            h3 = h.reshape(bsz, lq, D_MODEL)
            h3 = _distill(h3, cp["conv_w"], cp["conv_b"], cp["bn_g"], cp["bn_b"])
            lq = lq // 2
            h = h3.reshape(bsz * lq, D_MODEL)

    h3 = h.reshape(bsz, lq, D_MODEL)
    return _head(
        h3, x, p["norm_g"], p["norm_b"], p["head_w"], p["head_b"],
        p["skip_w"], p["skip_b"],
    )
